# baseline (speedup 1.0000x reference)
import jax
import jax.numpy as jnp
from jax.experimental import pallas as pl


def _head_body(h_ref, w2_ref, b2_ref, w1_ref, b1_ref, w0_ref, b0_ref, o_ref):
    h = h_ref[...]
    h = jnp.maximum(h @ w2_ref[...] + b2_ref[...], 0.0)
    h = jnp.maximum(h @ w1_ref[...] + b1_ref[...], 0.0)
    h = jnp.maximum(h @ w0_ref[...] + b0_ref[...], 0.0)
    o_ref[...] = h


def _conv_norm(x, edge_index, edge_weight, W, b, gamma, beta):
    n = x.shape[0]
    loop = jnp.arange(n, dtype=edge_index.dtype)
    src = jnp.concatenate([edge_index[0], loop])
    dst = jnp.concatenate([edge_index[1], loop])
    w = jnp.concatenate([edge_weight, jnp.ones((n,), x.dtype)])
    h = x @ W + b
    deg = jax.ops.segment_sum(w, dst, num_segments=n)
    dinv = jnp.where(deg > 0, jax.lax.rsqrt(jnp.maximum(deg, 1e-12)), 0.0)
    norm = dinv[src] * w * dinv[dst]
    out = jax.ops.segment_sum(h[src] * norm[:, None], dst, num_segments=n)
    mu = out.mean(axis=0)
    var = out.var(axis=0)
    return (out - mu) * jax.lax.rsqrt(var + 1e-5) * gamma + beta


def kernel(x, edge_index, edge_weight, batch, Wc0, bc0, g0, be0, Wc1, bc1, g1, be1,
           Wl2, bl2, Wl1, bl1, Wl0, bl0):
    h = jax.nn.relu(_conv_norm(x, edge_index, edge_weight, Wc0, bc0, g0, be0))
    h = jax.nn.relu(_conv_norm(h, edge_index, edge_weight, Wc1, bc1, g1, be1))
    G = 16
    sums = jax.ops.segment_sum(h, batch, num_segments=G)
    cnt = jax.ops.segment_sum(jnp.ones((h.shape[0],), h.dtype), batch, num_segments=G)
    hp = sums / jnp.maximum(cnt, 1.0)[:, None]
    out = pl.pallas_call(
        _head_body,
        out_shape=jax.ShapeDtypeStruct((G, Wl0.shape[1]), jnp.float32),
    )(hp, Wl2, bl2, Wl1, bl1, Wl0, bl0)
    return out


# speedup vs baseline: 4.6508x; 4.6508x over previous
import jax
import jax.numpy as jnp
from jax import lax
from jax.experimental import pallas as pl
from jax.experimental.pallas import tpu as pltpu
from jax.experimental.pallas import tpu_sc as plsc

N = 10000
E = 320000
G = 16
NC = 2
NS = 16
NW = NC * NS
EW = E // NW
BQ = 80
QI = 5
BI = BQ * QI
ITERS = EW // BI
NPAD = 10240
NT = NPAD // NS
NZ = NT // 5


def _sc_mesh():
    return plsc.VectorSubcoreMesh(core_axis_name="c", subcore_axis_name="s",
                                  num_cores=NC, num_subcores=NS)


def _agg_body(h_h, src1, dst1, wflat, out, acc, stage, sidx, wv, rows, *didx):
    c = lax.axis_index("c")
    s = lax.axis_index("s")
    wid = c * NS + s
    zf = jnp.zeros((16,), jnp.float32)

    def zloop(r, _):
        for k in range(8):
            stage.at[r][pl.ds(k * 16, 16)] = zf
        return 0
    lax.fori_loop(0, NZ, zloop, 0)
    for k in range(5):
        pltpu.sync_copy(stage, acc.at[pl.ds(s * NT + k * NZ, NZ)])
    plsc.subcore_barrier()

    def body(it, _):
        base = wid * EW + it * BI
        pltpu.sync_copy(src1.at[pl.ds(base, BI)], sidx)
        pltpu.sync_copy(wflat.at[pl.ds(base * 16, BI * 16)], wv)
        for q in range(QI):
            pltpu.sync_copy(dst1.at[pl.ds(base + q * BQ, BQ)], didx[q])
            pltpu.sync_copy(h_h.at[sidx.at[pl.ds(q * BQ, BQ)]], rows)

            def scale(j, _):
                cfb = wv[pl.ds((q * BQ + j) * 16, 16)]
                rr = rows.at[j]
                for k in range(8):
                    sl = pl.ds(k * 16, 16)
                    rr[sl] = rr[sl] * cfb
                return 0
            lax.fori_loop(0, BQ, scale, 0)
            pltpu.sync_copy(rows, acc.at[didx[q]], add=True)
        return 0
    lax.fori_loop(0, ITERS, body, 0)
    plsc.subcore_barrier()
    for k in range(5):
        pltpu.sync_copy(acc.at[pl.ds(s * NT + k * NZ, NZ)], stage)
        pltpu.sync_copy(stage, out.at[c, pl.ds(s * NT + k * NZ, NZ)])


def _agg_call(h_chunk, src1, dst1, wflat):
    return pl.kernel(
        _agg_body,
        out_type=jax.ShapeDtypeStruct((NC, NPAD, 128), jnp.float32),
        mesh=_sc_mesh(),
        scratch_types=[
            pltpu.VMEM_SHARED((NPAD, 128), jnp.float32),
            pltpu.VMEM((NZ, 128), jnp.float32),
            pltpu.VMEM((BI,), jnp.int32),
            pltpu.VMEM((BI * 16,), jnp.float32),
            pltpu.VMEM((BQ, 128), jnp.float32),
        ] + [pltpu.VMEM((BQ,), jnp.int32) for _ in range(QI)],
    )(h_chunk, src1, dst1, wflat)


_RB = 1000


def _w16_body(w_ref, o_ref):
    o_ref[...] = jnp.broadcast_to(w_ref[...], (8000, 16))


def _w16_call(w_col):
    return pl.pallas_call(
        _w16_body,
        grid=(E // 8000,),
        in_specs=[pl.BlockSpec((8000, 1), lambda i: (i, 0))],
        out_specs=pl.BlockSpec((8000, 16), lambda i: (i, 0)),
        out_shape=jax.ShapeDtypeStruct((E, 16), jnp.float32),
    )(w_col)


def _dinv_body(deg2_ref, dinv_ref):
    d = deg2_ref[0][:, 0:1] + deg2_ref[1][:, 0:1] + 1.0
    dinv_ref[...] = jnp.where(d > 0, lax.rsqrt(jnp.maximum(d, 1e-12)), 0.0)


def _dinv_call(deg2):
    return pl.pallas_call(
        _dinv_body,
        grid=(1,),
        in_specs=[pl.BlockSpec((NC, N, 128), lambda i: (0, 0, 0))],
        out_specs=pl.BlockSpec((N, 1), lambda i: (0, 0)),
        out_shape=jax.ShapeDtypeStruct((N, 1), jnp.float32),
    )(deg2)


def _mm0_body(x_ref, w_ref, b_ref, d_ref, oa_ref, ob_ref):
    h = (jnp.dot(x_ref[...], w_ref[...], preferred_element_type=jnp.float32)
         + b_ref[...]) * d_ref[...]
    oa_ref[...] = h[:, :128]
    ob_ref[...] = h[:, 128:]


def _mm0_call(x, W, b2, dinv):
    d_out = W.shape[1]
    return pl.pallas_call(
        _mm0_body,
        grid=(N // _RB,),
        in_specs=[
            pl.BlockSpec((_RB, x.shape[1]), lambda i: (i, 0)),
            pl.BlockSpec(W.shape, lambda i: (0, 0)),
            pl.BlockSpec((1, d_out), lambda i: (0, 0)),
            pl.BlockSpec((_RB, 1), lambda i: (i, 0)),
        ],
        out_specs=[pl.BlockSpec((_RB, 128), lambda i: (i, 0)),
                   pl.BlockSpec((_RB, 128), lambda i: (i, 0))],
        out_shape=[jax.ShapeDtypeStruct((N, 128), jnp.float32),
                   jax.ShapeDtypeStruct((N, 128), jnp.float32)],
    )(x, W, b2, dinv)


def _asm0_body(pa_ref, pb_ref, ha_ref, hb_ref, d_ref,
               oa_ref, ob_ref, ssum_ref, ssq_ref):
    d = d_ref[...]
    oa = (pa_ref[0] + pa_ref[1] + ha_ref[...]) * d
    ob = (pb_ref[0] + pb_ref[1] + hb_ref[...]) * d
    oa_ref[...] = oa
    ob_ref[...] = ob

    @pl.when(pl.program_id(0) == 0)
    def _():
        ssum_ref[...] = jnp.zeros_like(ssum_ref)
        ssq_ref[...] = jnp.zeros_like(ssq_ref)
    ssum_ref[:, :128] += jnp.sum(oa, 0, keepdims=True)
    ssum_ref[:, 128:] += jnp.sum(ob, 0, keepdims=True)
    ssq_ref[:, :128] += jnp.sum(oa * oa, 0, keepdims=True)
    ssq_ref[:, 128:] += jnp.sum(ob * ob, 0, keepdims=True)


def _asm0_call(pa, pb, ha, hb, dinv):
    return pl.pallas_call(
        _asm0_body,
        grid=(N // _RB,),
        in_specs=[
            pl.BlockSpec((NC, _RB, 128), lambda i: (0, i, 0)),
            pl.BlockSpec((NC, _RB, 128), lambda i: (0, i, 0)),
            pl.BlockSpec((_RB, 128), lambda i: (i, 0)),
            pl.BlockSpec((_RB, 128), lambda i: (i, 0)),
            pl.BlockSpec((_RB, 1), lambda i: (i, 0)),
        ],
        out_specs=[pl.BlockSpec((_RB, 128), lambda i: (i, 0)),
                   pl.BlockSpec((_RB, 128), lambda i: (i, 0)),
                   pl.BlockSpec((1, 256), lambda i: (0, 0)),
                   pl.BlockSpec((1, 256), lambda i: (0, 0))],
        out_shape=[jax.ShapeDtypeStruct((N, 128), jnp.float32),
                   jax.ShapeDtypeStruct((N, 128), jnp.float32),
                   jax.ShapeDtypeStruct((1, 256), jnp.float32),
                   jax.ShapeDtypeStruct((1, 256), jnp.float32)],
    )(pa, pb, ha, hb, dinv)


def _bnmm_body(oa_ref, ob_ref, ssum_ref, ssq_ref, g_ref, be_ref,
               w_ref, b_ref, d_ref, *out_refs):
    mu = ssum_ref[...] / N
    var = ssq_ref[...] / N - mu * mu
    sc = g_ref[...] * lax.rsqrt(var + 1e-5)
    t = be_ref[...] - mu * sc
    z = jnp.concatenate([oa_ref[...], ob_ref[...]], axis=1)
    a = jnp.maximum(z * sc + t, 0.0)
    h = (jnp.dot(a, w_ref[...], preferred_element_type=jnp.float32)
         + b_ref[...]) * d_ref[...]
    for k, o_ref in enumerate(out_refs):
        o_ref[...] = h[:, k * 128:(k + 1) * 128]


def _bnmm_call(oa, ob, ssum, ssq, g2, be2, W, b2, dinv):
    d_in, d_out = W.shape
    n_out = d_out // 128
    return pl.pallas_call(
        _bnmm_body,
        grid=(N // _RB,),
        in_specs=[
            pl.BlockSpec((_RB, 128), lambda i: (i, 0)),
            pl.BlockSpec((_RB, 128), lambda i: (i, 0)),
            pl.BlockSpec((1, d_in), lambda i: (0, 0)),
            pl.BlockSpec((1, d_in), lambda i: (0, 0)),
            pl.BlockSpec((1, d_in), lambda i: (0, 0)),
            pl.BlockSpec((1, d_in), lambda i: (0, 0)),
            pl.BlockSpec(W.shape, lambda i: (0, 0)),
            pl.BlockSpec((1, d_out), lambda i: (0, 0)),
            pl.BlockSpec((_RB, 1), lambda i: (i, 0)),
        ],
        out_specs=[pl.BlockSpec((_RB, 128), lambda i: (i, 0))
                   for _ in range(n_out)],
        out_shape=[jax.ShapeDtypeStruct((N, 128), jnp.float32)
                   for _ in range(n_out)],
    )(oa, ob, ssum, ssq, g2, be2, W, b2, dinv)


def _asm1_body(q0, q1, q2, q3, h0, h1, h2, h3, d_ref,
               o_ref, ssum_ref, ssq_ref):
    d = d_ref[...]

    @pl.when(pl.program_id(0) == 0)
    def _():
        ssum_ref[...] = jnp.zeros_like(ssum_ref)
        ssq_ref[...] = jnp.zeros_like(ssq_ref)
    for k, (q, hh) in enumerate(zip((q0, q1, q2, q3), (h0, h1, h2, h3))):
        o = (q[0] + q[1] + hh[...]) * d
        o_ref[:, k * 128:(k + 1) * 128] = o
        ssum_ref[:, k * 128:(k + 1) * 128] += jnp.sum(o, 0, keepdims=True)
        ssq_ref[:, k * 128:(k + 1) * 128] += jnp.sum(o * o, 0, keepdims=True)


def _asm1_call(qs, hs, dinv):
    return pl.pallas_call(
        _asm1_body,
        grid=(N // _RB,),
        in_specs=[pl.BlockSpec((NC, _RB, 128), lambda i: (0, i, 0))] * 4
        + [pl.BlockSpec((_RB, 128), lambda i: (i, 0))] * 4
        + [pl.BlockSpec((_RB, 1), lambda i: (i, 0))],
        out_specs=[pl.BlockSpec((_RB, 512), lambda i: (i, 0)),
                   pl.BlockSpec((1, 512), lambda i: (0, 0)),
                   pl.BlockSpec((1, 512), lambda i: (0, 0))],
        out_shape=[jax.ShapeDtypeStruct((N, 512), jnp.float32),
                   jax.ShapeDtypeStruct((1, 512), jnp.float32),
                   jax.ShapeDtypeStruct((1, 512), jnp.float32)],
    )(*qs, *hs, dinv)


def _pool_body(o_ref, ssum_ref, ssq_ref, g_ref, be_ref, batch_ref,
               psum_ref, cnt_ref):
    mu = ssum_ref[...] / N
    var = ssq_ref[...] / N - mu * mu
    sc = g_ref[...] * lax.rsqrt(var + 1e-5)
    t = be_ref[...] - mu * sc
    z = jnp.maximum(o_ref[...] * sc + t, 0.0)
    ids = lax.broadcasted_iota(jnp.int32, (_RB, G), 1)
    m = (ids == jnp.broadcast_to(batch_ref[...], (_RB, G))).astype(jnp.float32)

    @pl.when(pl.program_id(0) == 0)
    def _():
        psum_ref[...] = jnp.zeros_like(psum_ref)
        cnt_ref[...] = jnp.zeros_like(cnt_ref)
    dn = (((0,), (0,)), ((), ()))
    psum_ref[...] += lax.dot_general(m, z, dn,
                                     preferred_element_type=jnp.float32)
    cnt_ref[...] += lax.dot_general(m, jnp.ones((_RB, 128), jnp.float32), dn,
                                    preferred_element_type=jnp.float32)


def _pool_call(out1, ssum, ssq, g2, be2, batch_col):
    return pl.pallas_call(
        _pool_body,
        grid=(N // _RB,),
        in_specs=[
            pl.BlockSpec((_RB, 512), lambda i: (i, 0)),
            pl.BlockSpec((1, 512), lambda i: (0, 0)),
            pl.BlockSpec((1, 512), lambda i: (0, 0)),
            pl.BlockSpec((1, 512), lambda i: (0, 0)),
            pl.BlockSpec((1, 512), lambda i: (0, 0)),
            pl.BlockSpec((_RB, 1), lambda i: (i, 0)),
        ],
        out_specs=[pl.BlockSpec((G, 512), lambda i: (0, 0)),
                   pl.BlockSpec((G, 128), lambda i: (0, 0))],
        out_shape=[jax.ShapeDtypeStruct((G, 512), jnp.float32),
                   jax.ShapeDtypeStruct((G, 128), jnp.float32)],
    )(out1, ssum, ssq, g2, be2, batch_col)


def _head_body(ps_ref, cnt_ref, w2_ref, b2_ref, w1_ref, b1_ref,
               w0_ref, b0_ref, o_ref):
    cnt = jnp.maximum(cnt_ref[:, 0:1], 1.0)
    h = ps_ref[...] / cnt
    h = jnp.maximum(jnp.dot(h, w2_ref[...], preferred_element_type=jnp.float32)
                    + b2_ref[...], 0.0)
    h = jnp.maximum(jnp.dot(h, w1_ref[...], preferred_element_type=jnp.float32)
                    + b1_ref[...], 0.0)
    h = jnp.maximum(jnp.dot(h, w0_ref[...], preferred_element_type=jnp.float32)
                    + b0_ref[...], 0.0)
    o_ref[...] = h


def _head_call(psum, cnt, Wl2, bl2, Wl1, bl1, Wl0, bl0):
    return pl.pallas_call(
        _head_body,
        out_shape=jax.ShapeDtypeStruct((G, Wl0.shape[1]), jnp.float32),
    )(psum, cnt, Wl2, bl2.reshape(1, -1), Wl1, bl1.reshape(1, -1),
      Wl0, bl0.reshape(1, -1))


def kernel(x, edge_index, edge_weight, batch, Wc0, bc0, g0, be0,
           Wc1, bc1, g1, be1, Wl2, bl2, Wl1, bl1, Wl0, bl0):
    src1 = edge_index[0]
    dst1 = edge_index[1]

    w16 = _w16_call(edge_weight.reshape(E, 1))
    wflat = w16.reshape(E * 16)
    deg2 = _agg_call(jnp.ones((N, 128), jnp.float32), src1, dst1, wflat)
    dinv = _dinv_call(deg2)

    h0a, h0b = _mm0_call(x, Wc0, bc0.reshape(1, -1), dinv)
    pa = _agg_call(h0a, src1, dst1, wflat)
    pb = _agg_call(h0b, src1, dst1, wflat)
    oa, ob, ssum0, ssq0 = _asm0_call(pa, pb, h0a, h0b, dinv)
    h1s = _bnmm_call(oa, ob, ssum0, ssq0, g0.reshape(1, -1),
                     be0.reshape(1, -1), Wc1, bc1.reshape(1, -1), dinv)

    qs = [_agg_call(h1s[k], src1, dst1, wflat) for k in range(4)]
    out1, ssum1, ssq1 = _asm1_call(qs, h1s, dinv)

    psum, cnt = _pool_call(out1, ssum1, ssq1, g1.reshape(1, -1),
                           be1.reshape(1, -1), batch.reshape(N, 1))
    return _head_call(psum, cnt, Wl2, bl2, Wl1, bl1, Wl0, bl0)


# speedup vs baseline: 5.1288x; 1.1028x over previous
import jax
import jax.numpy as jnp
from jax import lax
from jax.experimental import pallas as pl
from jax.experimental.pallas import tpu as pltpu
from jax.experimental.pallas import tpu_sc as plsc

N = 10000
E = 320000
G = 16
NC = 2
NS = 16
NW = NC * NS
EW = E // NW
BQ = 80
QI = 5
BI = BQ * QI
ITERS = EW // BI
NPAD = 10240
NT = NPAD // NS
NZ = NT // 5


def _sc_mesh():
    return plsc.VectorSubcoreMesh(core_axis_name="c", subcore_axis_name="s",
                                  num_cores=NC, num_subcores=NS)


def _agg_body(h_h, src1, dst1, wflat, out, acc, stage, sidx, wv,
              rows0, rows1, gs0, gs1, ss0, ss1, *didx):
    c = lax.axis_index("c")
    s = lax.axis_index("s")
    wid = c * NS + s
    zf = jnp.zeros((16,), jnp.float32)
    rows = (rows0, rows1)
    gsem = (gs0, gs1)
    ssem = (ss0, ss1)

    def zloop(r, _):
        for k in range(8):
            stage.at[r][pl.ds(k * 16, 16)] = zf
        return 0
    lax.fori_loop(0, NZ, zloop, 0)
    for k in range(5):
        pltpu.sync_copy(stage, acc.at[pl.ds(s * NT + k * NZ, NZ)])
    plsc.subcore_barrier()

    def body(it, _):
        base = wid * EW + it * BI
        pltpu.sync_copy(src1.at[pl.ds(base, BI)], sidx)
        pltpu.sync_copy(wflat.at[pl.ds(base * 16, BI * 16)], wv)
        for q in range(QI):
            pltpu.sync_copy(dst1.at[pl.ds(base + q * BQ, BQ)], didx[q])
        gathers = [None] * QI
        scatters = [None] * QI
        gathers[0] = pltpu.async_copy(h_h.at[sidx.at[pl.ds(0, BQ)]],
                                      rows[0], gsem[0])
        for q in range(QI):
            cur, nxt = q % 2, (q + 1) % 2
            gathers[q].wait()

            def scale(j, _):
                cfb = wv[pl.ds((q * BQ + j) * 16, 16)]
                rr = rows[cur].at[j]
                for k in range(8):
                    sl = pl.ds(k * 16, 16)
                    rr[sl] = rr[sl] * cfb
                return 0
            lax.fori_loop(0, BQ, scale, 0)
            scatters[q] = pltpu.async_copy(rows[cur], acc.at[didx[q]],
                                           ssem[cur], add=True)
            if q + 1 < QI:
                if q >= 1:
                    scatters[q - 1].wait()
                gathers[q + 1] = pltpu.async_copy(
                    h_h.at[sidx.at[pl.ds((q + 1) * BQ, BQ)]],
                    rows[nxt], gsem[nxt])
        scatters[QI - 2].wait()
        scatters[QI - 1].wait()
        return 0
    lax.fori_loop(0, ITERS, body, 0)
    plsc.subcore_barrier()
    for k in range(5):
        pltpu.sync_copy(acc.at[pl.ds(s * NT + k * NZ, NZ)], stage)
        pltpu.sync_copy(stage, out.at[c, pl.ds(s * NT + k * NZ, NZ)])


def _agg_call(h_chunk, src1, dst1, wflat):
    return pl.kernel(
        _agg_body,
        out_type=jax.ShapeDtypeStruct((NC, NPAD, 128), jnp.float32),
        mesh=_sc_mesh(),
        scratch_types=[
            pltpu.VMEM_SHARED((NPAD, 128), jnp.float32),
            pltpu.VMEM((NZ, 128), jnp.float32),
            pltpu.VMEM((BI,), jnp.int32),
            pltpu.VMEM((BI * 16,), jnp.float32),
            pltpu.VMEM((BQ, 128), jnp.float32),
            pltpu.VMEM((BQ, 128), jnp.float32),
            pltpu.SemaphoreType.DMA,
            pltpu.SemaphoreType.DMA,
            pltpu.SemaphoreType.DMA,
            pltpu.SemaphoreType.DMA,
        ] + [pltpu.VMEM((BQ,), jnp.int32) for _ in range(QI)],
    )(h_chunk, src1, dst1, wflat)


_RB = 1000


def _w16_body(w_ref, o_ref):
    o_ref[...] = jnp.broadcast_to(w_ref[...], (8000, 16))


def _w16_call(w_col):
    return pl.pallas_call(
        _w16_body,
        grid=(E // 8000,),
        in_specs=[pl.BlockSpec((8000, 1), lambda i: (i, 0))],
        out_specs=pl.BlockSpec((8000, 16), lambda i: (i, 0)),
        out_shape=jax.ShapeDtypeStruct((E, 16), jnp.float32),
    )(w_col)


def _dinv_body(deg2_ref, dinv_ref):
    d = deg2_ref[0][:, 0:1] + deg2_ref[1][:, 0:1] + 1.0
    dinv_ref[...] = jnp.where(d > 0, lax.rsqrt(jnp.maximum(d, 1e-12)), 0.0)


def _dinv_call(deg2):
    return pl.pallas_call(
        _dinv_body,
        grid=(1,),
        in_specs=[pl.BlockSpec((NC, N, 128), lambda i: (0, 0, 0))],
        out_specs=pl.BlockSpec((N, 1), lambda i: (0, 0)),
        out_shape=jax.ShapeDtypeStruct((N, 1), jnp.float32),
    )(deg2)


def _mm0_body(x_ref, w_ref, b_ref, d_ref, oa_ref, ob_ref):
    h = (jnp.dot(x_ref[...], w_ref[...], preferred_element_type=jnp.float32)
         + b_ref[...]) * d_ref[...]
    oa_ref[...] = h[:, :128]
    ob_ref[...] = h[:, 128:]


def _mm0_call(x, W, b2, dinv):
    d_out = W.shape[1]
    return pl.pallas_call(
        _mm0_body,
        grid=(N // _RB,),
        in_specs=[
            pl.BlockSpec((_RB, x.shape[1]), lambda i: (i, 0)),
            pl.BlockSpec(W.shape, lambda i: (0, 0)),
            pl.BlockSpec((1, d_out), lambda i: (0, 0)),
            pl.BlockSpec((_RB, 1), lambda i: (i, 0)),
        ],
        out_specs=[pl.BlockSpec((_RB, 128), lambda i: (i, 0)),
                   pl.BlockSpec((_RB, 128), lambda i: (i, 0))],
        out_shape=[jax.ShapeDtypeStruct((N, 128), jnp.float32),
                   jax.ShapeDtypeStruct((N, 128), jnp.float32)],
    )(x, W, b2, dinv)


def _asm0_body(pa_ref, pb_ref, ha_ref, hb_ref, d_ref,
               oa_ref, ob_ref, ssum_ref, ssq_ref):
    d = d_ref[...]
    oa = (pa_ref[0] + pa_ref[1] + ha_ref[...]) * d
    ob = (pb_ref[0] + pb_ref[1] + hb_ref[...]) * d
    oa_ref[...] = oa
    ob_ref[...] = ob

    @pl.when(pl.program_id(0) == 0)
    def _():
        ssum_ref[...] = jnp.zeros_like(ssum_ref)
        ssq_ref[...] = jnp.zeros_like(ssq_ref)
    ssum_ref[:, :128] += jnp.sum(oa, 0, keepdims=True)
    ssum_ref[:, 128:] += jnp.sum(ob, 0, keepdims=True)
    ssq_ref[:, :128] += jnp.sum(oa * oa, 0, keepdims=True)
    ssq_ref[:, 128:] += jnp.sum(ob * ob, 0, keepdims=True)


def _asm0_call(pa, pb, ha, hb, dinv):
    return pl.pallas_call(
        _asm0_body,
        grid=(N // _RB,),
        in_specs=[
            pl.BlockSpec((NC, _RB, 128), lambda i: (0, i, 0)),
            pl.BlockSpec((NC, _RB, 128), lambda i: (0, i, 0)),
            pl.BlockSpec((_RB, 128), lambda i: (i, 0)),
            pl.BlockSpec((_RB, 128), lambda i: (i, 0)),
            pl.BlockSpec((_RB, 1), lambda i: (i, 0)),
        ],
        out_specs=[pl.BlockSpec((_RB, 128), lambda i: (i, 0)),
                   pl.BlockSpec((_RB, 128), lambda i: (i, 0)),
                   pl.BlockSpec((1, 256), lambda i: (0, 0)),
                   pl.BlockSpec((1, 256), lambda i: (0, 0))],
        out_shape=[jax.ShapeDtypeStruct((N, 128), jnp.float32),
                   jax.ShapeDtypeStruct((N, 128), jnp.float32),
                   jax.ShapeDtypeStruct((1, 256), jnp.float32),
                   jax.ShapeDtypeStruct((1, 256), jnp.float32)],
    )(pa, pb, ha, hb, dinv)


def _bnmm_body(oa_ref, ob_ref, ssum_ref, ssq_ref, g_ref, be_ref,
               w_ref, b_ref, d_ref, *out_refs):
    mu = ssum_ref[...] / N
    var = ssq_ref[...] / N - mu * mu
    sc = g_ref[...] * lax.rsqrt(var + 1e-5)
    t = be_ref[...] - mu * sc
    z = jnp.concatenate([oa_ref[...], ob_ref[...]], axis=1)
    a = jnp.maximum(z * sc + t, 0.0)
    h = (jnp.dot(a, w_ref[...], preferred_element_type=jnp.float32)
         + b_ref[...]) * d_ref[...]
    for k, o_ref in enumerate(out_refs):
        o_ref[...] = h[:, k * 128:(k + 1) * 128]


def _bnmm_call(oa, ob, ssum, ssq, g2, be2, W, b2, dinv):
    d_in, d_out = W.shape
    n_out = d_out // 128
    return pl.pallas_call(
        _bnmm_body,
        grid=(N // _RB,),
        in_specs=[
            pl.BlockSpec((_RB, 128), lambda i: (i, 0)),
            pl.BlockSpec((_RB, 128), lambda i: (i, 0)),
            pl.BlockSpec((1, d_in), lambda i: (0, 0)),
            pl.BlockSpec((1, d_in), lambda i: (0, 0)),
            pl.BlockSpec((1, d_in), lambda i: (0, 0)),
            pl.BlockSpec((1, d_in), lambda i: (0, 0)),
            pl.BlockSpec(W.shape, lambda i: (0, 0)),
            pl.BlockSpec((1, d_out), lambda i: (0, 0)),
            pl.BlockSpec((_RB, 1), lambda i: (i, 0)),
        ],
        out_specs=[pl.BlockSpec((_RB, 128), lambda i: (i, 0))
                   for _ in range(n_out)],
        out_shape=[jax.ShapeDtypeStruct((N, 128), jnp.float32)
                   for _ in range(n_out)],
    )(oa, ob, ssum, ssq, g2, be2, W, b2, dinv)


def _asm1_body(q0, q1, q2, q3, h0, h1, h2, h3, d_ref,
               o_ref, ssum_ref, ssq_ref):
    d = d_ref[...]

    @pl.when(pl.program_id(0) == 0)
    def _():
        ssum_ref[...] = jnp.zeros_like(ssum_ref)
        ssq_ref[...] = jnp.zeros_like(ssq_ref)
    for k, (q, hh) in enumerate(zip((q0, q1, q2, q3), (h0, h1, h2, h3))):
        o = (q[0] + q[1] + hh[...]) * d
        o_ref[:, k * 128:(k + 1) * 128] = o
        ssum_ref[:, k * 128:(k + 1) * 128] += jnp.sum(o, 0, keepdims=True)
        ssq_ref[:, k * 128:(k + 1) * 128] += jnp.sum(o * o, 0, keepdims=True)


def _asm1_call(qs, hs, dinv):
    return pl.pallas_call(
        _asm1_body,
        grid=(N // _RB,),
        in_specs=[pl.BlockSpec((NC, _RB, 128), lambda i: (0, i, 0))] * 4
        + [pl.BlockSpec((_RB, 128), lambda i: (i, 0))] * 4
        + [pl.BlockSpec((_RB, 1), lambda i: (i, 0))],
        out_specs=[pl.BlockSpec((_RB, 512), lambda i: (i, 0)),
                   pl.BlockSpec((1, 512), lambda i: (0, 0)),
                   pl.BlockSpec((1, 512), lambda i: (0, 0))],
        out_shape=[jax.ShapeDtypeStruct((N, 512), jnp.float32),
                   jax.ShapeDtypeStruct((1, 512), jnp.float32),
                   jax.ShapeDtypeStruct((1, 512), jnp.float32)],
    )(*qs, *hs, dinv)


def _pool_body(o_ref, ssum_ref, ssq_ref, g_ref, be_ref, batch_ref,
               psum_ref, cnt_ref):
    mu = ssum_ref[...] / N
    var = ssq_ref[...] / N - mu * mu
    sc = g_ref[...] * lax.rsqrt(var + 1e-5)
    t = be_ref[...] - mu * sc
    z = jnp.maximum(o_ref[...] * sc + t, 0.0)
    ids = lax.broadcasted_iota(jnp.int32, (_RB, G), 1)
    m = (ids == jnp.broadcast_to(batch_ref[...], (_RB, G))).astype(jnp.float32)

    @pl.when(pl.program_id(0) == 0)
    def _():
        psum_ref[...] = jnp.zeros_like(psum_ref)
        cnt_ref[...] = jnp.zeros_like(cnt_ref)
    dn = (((0,), (0,)), ((), ()))
    psum_ref[...] += lax.dot_general(m, z, dn,
                                     preferred_element_type=jnp.float32)
    cnt_ref[...] += lax.dot_general(m, jnp.ones((_RB, 128), jnp.float32), dn,
                                    preferred_element_type=jnp.float32)


def _pool_call(out1, ssum, ssq, g2, be2, batch_col):
    return pl.pallas_call(
        _pool_body,
        grid=(N // _RB,),
        in_specs=[
            pl.BlockSpec((_RB, 512), lambda i: (i, 0)),
            pl.BlockSpec((1, 512), lambda i: (0, 0)),
            pl.BlockSpec((1, 512), lambda i: (0, 0)),
            pl.BlockSpec((1, 512), lambda i: (0, 0)),
            pl.BlockSpec((1, 512), lambda i: (0, 0)),
            pl.BlockSpec((_RB, 1), lambda i: (i, 0)),
        ],
        out_specs=[pl.BlockSpec((G, 512), lambda i: (0, 0)),
                   pl.BlockSpec((G, 128), lambda i: (0, 0))],
        out_shape=[jax.ShapeDtypeStruct((G, 512), jnp.float32),
                   jax.ShapeDtypeStruct((G, 128), jnp.float32)],
    )(out1, ssum, ssq, g2, be2, batch_col)


def _head_body(ps_ref, cnt_ref, w2_ref, b2_ref, w1_ref, b1_ref,
               w0_ref, b0_ref, o_ref):
    cnt = jnp.maximum(cnt_ref[:, 0:1], 1.0)
    h = ps_ref[...] / cnt
    h = jnp.maximum(jnp.dot(h, w2_ref[...], preferred_element_type=jnp.float32)
                    + b2_ref[...], 0.0)
    h = jnp.maximum(jnp.dot(h, w1_ref[...], preferred_element_type=jnp.float32)
                    + b1_ref[...], 0.0)
    h = jnp.maximum(jnp.dot(h, w0_ref[...], preferred_element_type=jnp.float32)
                    + b0_ref[...], 0.0)
    o_ref[...] = h


def _head_call(psum, cnt, Wl2, bl2, Wl1, bl1, Wl0, bl0):
    return pl.pallas_call(
        _head_body,
        out_shape=jax.ShapeDtypeStruct((G, Wl0.shape[1]), jnp.float32),
    )(psum, cnt, Wl2, bl2.reshape(1, -1), Wl1, bl1.reshape(1, -1),
      Wl0, bl0.reshape(1, -1))


def kernel(x, edge_index, edge_weight, batch, Wc0, bc0, g0, be0,
           Wc1, bc1, g1, be1, Wl2, bl2, Wl1, bl1, Wl0, bl0):
    src1 = edge_index[0]
    dst1 = edge_index[1]

    w16 = _w16_call(edge_weight.reshape(E, 1))
    wflat = w16.reshape(E * 16)
    deg2 = _agg_call(jnp.ones((N, 128), jnp.float32), src1, dst1, wflat)
    dinv = _dinv_call(deg2)

    h0a, h0b = _mm0_call(x, Wc0, bc0.reshape(1, -1), dinv)
    pa = _agg_call(h0a, src1, dst1, wflat)
    pb = _agg_call(h0b, src1, dst1, wflat)
    oa, ob, ssum0, ssq0 = _asm0_call(pa, pb, h0a, h0b, dinv)
    h1s = _bnmm_call(oa, ob, ssum0, ssq0, g0.reshape(1, -1),
                     be0.reshape(1, -1), Wc1, bc1.reshape(1, -1), dinv)

    qs = [_agg_call(h1s[k], src1, dst1, wflat) for k in range(4)]
    out1, ssum1, ssq1 = _asm1_call(qs, h1s, dinv)

    psum, cnt = _pool_call(out1, ssum1, ssq1, g1.reshape(1, -1),
                           be1.reshape(1, -1), batch.reshape(N, 1))
    return _head_call(psum, cnt, Wl2, bl2, Wl1, bl1, Wl0, bl0)


# speedup vs baseline: 6.8387x; 1.3334x over previous
import jax
import jax.numpy as jnp
from jax import lax
from jax.experimental import pallas as pl
from jax.experimental.pallas import tpu as pltpu
from jax.experimental.pallas import tpu_sc as plsc

N = 10000
E = 320000
G = 16
NC = 2
NS = 16
NW = NC * NS
EW = E // NW
BQ = 80
QI = 5
BI = BQ * QI
ITERS = EW // BI
NPAD = 10240
NT = NPAD // NS
NZ = NT // 5


def _sc_mesh():
    return plsc.VectorSubcoreMesh(core_axis_name="c", subcore_axis_name="s",
                                  num_cores=NC, num_subcores=NS)


def _agg_body(h_h, src1, dst1, wflat, out, acc, stage, sidx, wv,
              rows0, rows1, gs0, gs1, ss0, ss1, *didx):
    c = lax.axis_index("c")
    s = lax.axis_index("s")
    wid = c * NS + s
    zf = jnp.zeros((16,), jnp.float32)
    rows = (rows0, rows1)
    gsem = (gs0, gs1)
    ssem = (ss0, ss1)

    def zloop(r, _):
        for k in range(8):
            stage.at[r][pl.ds(k * 16, 16)] = zf
        return 0
    lax.fori_loop(0, NZ, zloop, 0)
    for k in range(5):
        pltpu.sync_copy(stage, acc.at[pl.ds(s * NT + k * NZ, NZ)])
    plsc.subcore_barrier()

    def body(it, _):
        base = wid * EW + it * BI
        pltpu.sync_copy(src1.at[pl.ds(base, BI)], sidx)
        pltpu.sync_copy(wflat.at[pl.ds(base, BI)], wv)
        for q in range(QI):
            pltpu.sync_copy(dst1.at[pl.ds(base + q * BQ, BQ)], didx[q])
        gathers = [None] * QI
        scatters = [None] * QI
        gathers[0] = pltpu.async_copy(h_h.at[sidx.at[pl.ds(0, BQ)]],
                                      rows[0], gsem[0])
        for q in range(QI):
            cur, nxt = q % 2, (q + 1) % 2
            gathers[q].wait()

            def scale(g, _):
                wpk = wv[pl.ds(q * BQ + g * 16, 16)]
                for l in range(16):
                    cfb = lax.gather(
                        wpk, jnp.full((16, 1), l, jnp.int32),
                        lax.GatherDimensionNumbers(
                            offset_dims=(), collapsed_slice_dims=(0,),
                            start_index_map=(0,)),
                        (1,), mode=lax.GatherScatterMode.PROMISE_IN_BOUNDS)
                    rr = rows[cur].at[g * 16 + l]
                    for k in range(8):
                        sl = pl.ds(k * 16, 16)
                        rr[sl] = rr[sl] * cfb
                return 0
            lax.fori_loop(0, BQ // 16, scale, 0)
            scatters[q] = pltpu.async_copy(rows[cur], acc.at[didx[q]],
                                           ssem[cur], add=True)
            if q + 1 < QI:
                if q >= 1:
                    scatters[q - 1].wait()
                gathers[q + 1] = pltpu.async_copy(
                    h_h.at[sidx.at[pl.ds((q + 1) * BQ, BQ)]],
                    rows[nxt], gsem[nxt])
        scatters[QI - 2].wait()
        scatters[QI - 1].wait()
        return 0
    lax.fori_loop(0, ITERS, body, 0)
    plsc.subcore_barrier()
    for k in range(5):
        pltpu.sync_copy(acc.at[pl.ds(s * NT + k * NZ, NZ)], stage)
        pltpu.sync_copy(stage, out.at[c, pl.ds(s * NT + k * NZ, NZ)])


def _agg_call(h_chunk, src1, dst1, wflat):
    return pl.kernel(
        _agg_body,
        out_type=jax.ShapeDtypeStruct((NC, NPAD, 128), jnp.float32),
        mesh=_sc_mesh(),
        scratch_types=[
            pltpu.VMEM_SHARED((NPAD, 128), jnp.float32),
            pltpu.VMEM((NZ, 128), jnp.float32),
            pltpu.VMEM((BI,), jnp.int32),
            pltpu.VMEM((BI,), jnp.float32),
            pltpu.VMEM((BQ, 128), jnp.float32),
            pltpu.VMEM((BQ, 128), jnp.float32),
            pltpu.SemaphoreType.DMA,
            pltpu.SemaphoreType.DMA,
            pltpu.SemaphoreType.DMA,
            pltpu.SemaphoreType.DMA,
        ] + [pltpu.VMEM((BQ,), jnp.int32) for _ in range(QI)],
    )(h_chunk, src1, dst1, wflat)


_RB = 1000


def _dinv_body(deg2_ref, dinv_ref):
    d = deg2_ref[0][:, 0:1] + deg2_ref[1][:, 0:1] + 1.0
    dinv_ref[...] = jnp.where(d > 0, lax.rsqrt(jnp.maximum(d, 1e-12)), 0.0)


def _dinv_call(deg2):
    return pl.pallas_call(
        _dinv_body,
        grid=(1,),
        in_specs=[pl.BlockSpec((NC, N, 128), lambda i: (0, 0, 0))],
        out_specs=pl.BlockSpec((N, 1), lambda i: (0, 0)),
        out_shape=jax.ShapeDtypeStruct((N, 1), jnp.float32),
    )(deg2)


def _mm0_body(x_ref, w_ref, b_ref, d_ref, oa_ref, ob_ref):
    h = (jnp.dot(x_ref[...], w_ref[...], preferred_element_type=jnp.float32)
         + b_ref[...]) * d_ref[...]
    oa_ref[...] = h[:, :128]
    ob_ref[...] = h[:, 128:]


def _mm0_call(x, W, b2, dinv):
    d_out = W.shape[1]
    return pl.pallas_call(
        _mm0_body,
        grid=(N // _RB,),
        in_specs=[
            pl.BlockSpec((_RB, x.shape[1]), lambda i: (i, 0)),
            pl.BlockSpec(W.shape, lambda i: (0, 0)),
            pl.BlockSpec((1, d_out), lambda i: (0, 0)),
            pl.BlockSpec((_RB, 1), lambda i: (i, 0)),
        ],
        out_specs=[pl.BlockSpec((_RB, 128), lambda i: (i, 0)),
                   pl.BlockSpec((_RB, 128), lambda i: (i, 0))],
        out_shape=[jax.ShapeDtypeStruct((N, 128), jnp.float32),
                   jax.ShapeDtypeStruct((N, 128), jnp.float32)],
    )(x, W, b2, dinv)


def _asm0_body(pa_ref, pb_ref, ha_ref, hb_ref, d_ref,
               oa_ref, ob_ref, ssum_ref, ssq_ref):
    d = d_ref[...]
    oa = (pa_ref[0] + pa_ref[1] + ha_ref[...]) * d
    ob = (pb_ref[0] + pb_ref[1] + hb_ref[...]) * d
    oa_ref[...] = oa
    ob_ref[...] = ob

    @pl.when(pl.program_id(0) == 0)
    def _():
        ssum_ref[...] = jnp.zeros_like(ssum_ref)
        ssq_ref[...] = jnp.zeros_like(ssq_ref)
    ssum_ref[:, :128] += jnp.sum(oa, 0, keepdims=True)
    ssum_ref[:, 128:] += jnp.sum(ob, 0, keepdims=True)
    ssq_ref[:, :128] += jnp.sum(oa * oa, 0, keepdims=True)
    ssq_ref[:, 128:] += jnp.sum(ob * ob, 0, keepdims=True)


def _asm0_call(pa, pb, ha, hb, dinv):
    return pl.pallas_call(
        _asm0_body,
        grid=(N // _RB,),
        in_specs=[
            pl.BlockSpec((NC, _RB, 128), lambda i: (0, i, 0)),
            pl.BlockSpec((NC, _RB, 128), lambda i: (0, i, 0)),
            pl.BlockSpec((_RB, 128), lambda i: (i, 0)),
            pl.BlockSpec((_RB, 128), lambda i: (i, 0)),
            pl.BlockSpec((_RB, 1), lambda i: (i, 0)),
        ],
        out_specs=[pl.BlockSpec((_RB, 128), lambda i: (i, 0)),
                   pl.BlockSpec((_RB, 128), lambda i: (i, 0)),
                   pl.BlockSpec((1, 256), lambda i: (0, 0)),
                   pl.BlockSpec((1, 256), lambda i: (0, 0))],
        out_shape=[jax.ShapeDtypeStruct((N, 128), jnp.float32),
                   jax.ShapeDtypeStruct((N, 128), jnp.float32),
                   jax.ShapeDtypeStruct((1, 256), jnp.float32),
                   jax.ShapeDtypeStruct((1, 256), jnp.float32)],
    )(pa, pb, ha, hb, dinv)


def _bnmm_body(oa_ref, ob_ref, ssum_ref, ssq_ref, g_ref, be_ref,
               w_ref, b_ref, d_ref, *out_refs):
    mu = ssum_ref[...] / N
    var = ssq_ref[...] / N - mu * mu
    sc = g_ref[...] * lax.rsqrt(var + 1e-5)
    t = be_ref[...] - mu * sc
    z = jnp.concatenate([oa_ref[...], ob_ref[...]], axis=1)
    a = jnp.maximum(z * sc + t, 0.0)
    h = (jnp.dot(a, w_ref[...], preferred_element_type=jnp.float32)
         + b_ref[...]) * d_ref[...]
    for k, o_ref in enumerate(out_refs):
        o_ref[...] = h[:, k * 128:(k + 1) * 128]


def _bnmm_call(oa, ob, ssum, ssq, g2, be2, W, b2, dinv):
    d_in, d_out = W.shape
    n_out = d_out // 128
    return pl.pallas_call(
        _bnmm_body,
        grid=(N // _RB,),
        in_specs=[
            pl.BlockSpec((_RB, 128), lambda i: (i, 0)),
            pl.BlockSpec((_RB, 128), lambda i: (i, 0)),
            pl.BlockSpec((1, d_in), lambda i: (0, 0)),
            pl.BlockSpec((1, d_in), lambda i: (0, 0)),
            pl.BlockSpec((1, d_in), lambda i: (0, 0)),
            pl.BlockSpec((1, d_in), lambda i: (0, 0)),
            pl.BlockSpec(W.shape, lambda i: (0, 0)),
            pl.BlockSpec((1, d_out), lambda i: (0, 0)),
            pl.BlockSpec((_RB, 1), lambda i: (i, 0)),
        ],
        out_specs=[pl.BlockSpec((_RB, 128), lambda i: (i, 0))
                   for _ in range(n_out)],
        out_shape=[jax.ShapeDtypeStruct((N, 128), jnp.float32)
                   for _ in range(n_out)],
    )(oa, ob, ssum, ssq, g2, be2, W, b2, dinv)


def _asm1_body(q0, q1, q2, q3, h0, h1, h2, h3, d_ref,
               o_ref, ssum_ref, ssq_ref):
    d = d_ref[...]

    @pl.when(pl.program_id(0) == 0)
    def _():
        ssum_ref[...] = jnp.zeros_like(ssum_ref)
        ssq_ref[...] = jnp.zeros_like(ssq_ref)
    for k, (q, hh) in enumerate(zip((q0, q1, q2, q3), (h0, h1, h2, h3))):
        o = (q[0] + q[1] + hh[...]) * d
        o_ref[:, k * 128:(k + 1) * 128] = o
        ssum_ref[:, k * 128:(k + 1) * 128] += jnp.sum(o, 0, keepdims=True)
        ssq_ref[:, k * 128:(k + 1) * 128] += jnp.sum(o * o, 0, keepdims=True)


def _asm1_call(qs, hs, dinv):
    return pl.pallas_call(
        _asm1_body,
        grid=(N // _RB,),
        in_specs=[pl.BlockSpec((NC, _RB, 128), lambda i: (0, i, 0))] * 4
        + [pl.BlockSpec((_RB, 128), lambda i: (i, 0))] * 4
        + [pl.BlockSpec((_RB, 1), lambda i: (i, 0))],
        out_specs=[pl.BlockSpec((_RB, 512), lambda i: (i, 0)),
                   pl.BlockSpec((1, 512), lambda i: (0, 0)),
                   pl.BlockSpec((1, 512), lambda i: (0, 0))],
        out_shape=[jax.ShapeDtypeStruct((N, 512), jnp.float32),
                   jax.ShapeDtypeStruct((1, 512), jnp.float32),
                   jax.ShapeDtypeStruct((1, 512), jnp.float32)],
    )(*qs, *hs, dinv)


def _pool_body(o_ref, ssum_ref, ssq_ref, g_ref, be_ref, batch_ref,
               psum_ref, cnt_ref):
    mu = ssum_ref[...] / N
    var = ssq_ref[...] / N - mu * mu
    sc = g_ref[...] * lax.rsqrt(var + 1e-5)
    t = be_ref[...] - mu * sc
    z = jnp.maximum(o_ref[...] * sc + t, 0.0)
    ids = lax.broadcasted_iota(jnp.int32, (_RB, G), 1)
    m = (ids == jnp.broadcast_to(batch_ref[...], (_RB, G))).astype(jnp.float32)

    @pl.when(pl.program_id(0) == 0)
    def _():
        psum_ref[...] = jnp.zeros_like(psum_ref)
        cnt_ref[...] = jnp.zeros_like(cnt_ref)
    dn = (((0,), (0,)), ((), ()))
    psum_ref[...] += lax.dot_general(m, z, dn,
                                     preferred_element_type=jnp.float32)
    cnt_ref[...] += lax.dot_general(m, jnp.ones((_RB, 128), jnp.float32), dn,
                                    preferred_element_type=jnp.float32)


def _pool_call(out1, ssum, ssq, g2, be2, batch_col):
    return pl.pallas_call(
        _pool_body,
        grid=(N // _RB,),
        in_specs=[
            pl.BlockSpec((_RB, 512), lambda i: (i, 0)),
            pl.BlockSpec((1, 512), lambda i: (0, 0)),
            pl.BlockSpec((1, 512), lambda i: (0, 0)),
            pl.BlockSpec((1, 512), lambda i: (0, 0)),
            pl.BlockSpec((1, 512), lambda i: (0, 0)),
            pl.BlockSpec((_RB, 1), lambda i: (i, 0)),
        ],
        out_specs=[pl.BlockSpec((G, 512), lambda i: (0, 0)),
                   pl.BlockSpec((G, 128), lambda i: (0, 0))],
        out_shape=[jax.ShapeDtypeStruct((G, 512), jnp.float32),
                   jax.ShapeDtypeStruct((G, 128), jnp.float32)],
    )(out1, ssum, ssq, g2, be2, batch_col)


def _head_body(ps_ref, cnt_ref, w2_ref, b2_ref, w1_ref, b1_ref,
               w0_ref, b0_ref, o_ref):
    cnt = jnp.maximum(cnt_ref[:, 0:1], 1.0)
    h = ps_ref[...] / cnt
    h = jnp.maximum(jnp.dot(h, w2_ref[...], preferred_element_type=jnp.float32)
                    + b2_ref[...], 0.0)
    h = jnp.maximum(jnp.dot(h, w1_ref[...], preferred_element_type=jnp.float32)
                    + b1_ref[...], 0.0)
    h = jnp.maximum(jnp.dot(h, w0_ref[...], preferred_element_type=jnp.float32)
                    + b0_ref[...], 0.0)
    o_ref[...] = h


def _head_call(psum, cnt, Wl2, bl2, Wl1, bl1, Wl0, bl0):
    return pl.pallas_call(
        _head_body,
        out_shape=jax.ShapeDtypeStruct((G, Wl0.shape[1]), jnp.float32),
    )(psum, cnt, Wl2, bl2.reshape(1, -1), Wl1, bl1.reshape(1, -1),
      Wl0, bl0.reshape(1, -1))


def kernel(x, edge_index, edge_weight, batch, Wc0, bc0, g0, be0,
           Wc1, bc1, g1, be1, Wl2, bl2, Wl1, bl1, Wl0, bl0):
    src1 = edge_index[0]
    dst1 = edge_index[1]

    wflat = edge_weight
    deg2 = _agg_call(jnp.ones((N, 128), jnp.float32), src1, dst1, wflat)
    dinv = _dinv_call(deg2)

    h0a, h0b = _mm0_call(x, Wc0, bc0.reshape(1, -1), dinv)
    pa = _agg_call(h0a, src1, dst1, wflat)
    pb = _agg_call(h0b, src1, dst1, wflat)
    oa, ob, ssum0, ssq0 = _asm0_call(pa, pb, h0a, h0b, dinv)
    h1s = _bnmm_call(oa, ob, ssum0, ssq0, g0.reshape(1, -1),
                     be0.reshape(1, -1), Wc1, bc1.reshape(1, -1), dinv)

    qs = [_agg_call(h1s[k], src1, dst1, wflat) for k in range(4)]
    out1, ssum1, ssq1 = _asm1_call(qs, h1s, dinv)

    psum, cnt = _pool_call(out1, ssum1, ssq1, g1.reshape(1, -1),
                           be1.reshape(1, -1), batch.reshape(N, 1))
    return _head_call(psum, cnt, Wl2, bl2, Wl1, bl1, Wl0, bl0)


# speedup vs baseline: 8.5998x; 1.2575x over previous
import jax
import jax.numpy as jnp
from jax import lax
from jax.experimental import pallas as pl
from jax.experimental.pallas import tpu as pltpu
from jax.experimental.pallas import tpu_sc as plsc

N = 10000
E = 320000
G = 16
NC = 2
NS = 16
NW = NC * NS
EW = E // NW
BQ = 80
QI = 5
BI = BQ * QI
ITERS = EW // BI
NPAD = 10240
NT = NPAD // NS
NZ = NT // 5


def _sc_mesh():
    return plsc.VectorSubcoreMesh(core_axis_name="c", subcore_axis_name="s",
                                  num_cores=NC, num_subcores=NS)


def _agg_body(h_h, src1, dst1, wflat, out, acc, stage, sidx, wv,
              rows0, rows1, gs0, gs1, ss0, ss1, psem, didx):
    c = lax.axis_index("c")
    s = lax.axis_index("s")
    wid = c * NS + s
    zf = jnp.zeros((16,), jnp.float32)
    rows = (rows0, rows1)
    gsem = (gs0, gs1)
    ssem = (ss0, ss1)

    def zloop(r, _):
        for k in range(8):
            stage.at[r][pl.ds(k * 16, 16)] = zf
        return 0
    lax.fori_loop(0, NZ, zloop, 0)
    for k in range(5):
        pltpu.sync_copy(stage, acc.at[pl.ds(s * NT + k * NZ, NZ)])
    plsc.subcore_barrier()

    def body(it, _):
        base = wid * EW + it * BI
        pf = [pltpu.async_copy(src1.at[pl.ds(base, BI)], sidx, psem),
              pltpu.async_copy(wflat.at[pl.ds(base, BI)], wv, psem),
              pltpu.async_copy(dst1.at[pl.ds(base, BI)], didx, psem)]
        for d in pf:
            d.wait()
        gathers = [None] * QI
        scatters = [None] * QI
        gathers[0] = pltpu.async_copy(h_h.at[sidx.at[pl.ds(0, BQ)]],
                                      rows[0], gsem[0])
        for q in range(QI):
            cur, nxt = q % 2, (q + 1) % 2
            gathers[q].wait()

            def scale(g, _):
                wpk = wv[pl.ds(q * BQ + g * 16, 16)]
                for l in range(16):
                    cfb = lax.gather(
                        wpk, jnp.full((16, 1), l, jnp.int32),
                        lax.GatherDimensionNumbers(
                            offset_dims=(), collapsed_slice_dims=(0,),
                            start_index_map=(0,)),
                        (1,), mode=lax.GatherScatterMode.PROMISE_IN_BOUNDS)
                    rr = rows[cur].at[g * 16 + l]
                    for k in range(8):
                        sl = pl.ds(k * 16, 16)
                        rr[sl] = rr[sl] * cfb
                return 0
            lax.fori_loop(0, BQ // 16, scale, 0)
            scatters[q] = pltpu.async_copy(
                rows[cur], acc.at[didx.at[pl.ds(q * BQ, BQ)]],
                ssem[cur], add=True)
            if q + 1 < QI:
                if q >= 1:
                    scatters[q - 1].wait()
                gathers[q + 1] = pltpu.async_copy(
                    h_h.at[sidx.at[pl.ds((q + 1) * BQ, BQ)]],
                    rows[nxt], gsem[nxt])
        scatters[QI - 2].wait()
        scatters[QI - 1].wait()
        return 0
    lax.fori_loop(0, ITERS, body, 0)
    plsc.subcore_barrier()
    for k in range(5):
        pltpu.sync_copy(acc.at[pl.ds(s * NT + k * NZ, NZ)], stage)
        pltpu.sync_copy(stage, out.at[c, pl.ds(s * NT + k * NZ, NZ)])


def _agg_call(h_chunk, src1, dst1, wflat):
    return pl.kernel(
        _agg_body,
        out_type=jax.ShapeDtypeStruct((NC, NPAD, 128), jnp.float32),
        mesh=_sc_mesh(),
        scratch_types=[
            pltpu.VMEM_SHARED((NPAD, 128), jnp.float32),
            pltpu.VMEM((NZ, 128), jnp.float32),
            pltpu.VMEM((BI,), jnp.int32),
            pltpu.VMEM((BI,), jnp.float32),
            pltpu.VMEM((BQ, 128), jnp.float32),
            pltpu.VMEM((BQ, 128), jnp.float32),
            pltpu.SemaphoreType.DMA,
            pltpu.SemaphoreType.DMA,
            pltpu.SemaphoreType.DMA,
            pltpu.SemaphoreType.DMA,
            pltpu.SemaphoreType.DMA,
            pltpu.VMEM((BI,), jnp.int32),
        ],
    )(h_chunk, src1, dst1, wflat)


_RB = 1000


def _dinv_body(deg2_ref, dinv_ref):
    d = deg2_ref[0][:, 0:1] + deg2_ref[1][:, 0:1] + 1.0
    dinv_ref[...] = jnp.where(d > 0, lax.rsqrt(jnp.maximum(d, 1e-12)), 0.0)


def _dinv_call(deg2):
    return pl.pallas_call(
        _dinv_body,
        grid=(1,),
        in_specs=[pl.BlockSpec((NC, N, 128), lambda i: (0, 0, 0))],
        out_specs=pl.BlockSpec((N, 1), lambda i: (0, 0)),
        out_shape=jax.ShapeDtypeStruct((N, 1), jnp.float32),
    )(deg2)


def _mm0_body(x_ref, w_ref, b_ref, d_ref, oa_ref, ob_ref):
    h = (jnp.dot(x_ref[...], w_ref[...], preferred_element_type=jnp.float32)
         + b_ref[...]) * d_ref[...]
    oa_ref[...] = h[:, :128]
    ob_ref[...] = h[:, 128:]


def _mm0_call(x, W, b2, dinv):
    d_out = W.shape[1]
    return pl.pallas_call(
        _mm0_body,
        grid=(N // _RB,),
        in_specs=[
            pl.BlockSpec((_RB, x.shape[1]), lambda i: (i, 0)),
            pl.BlockSpec(W.shape, lambda i: (0, 0)),
            pl.BlockSpec((1, d_out), lambda i: (0, 0)),
            pl.BlockSpec((_RB, 1), lambda i: (i, 0)),
        ],
        out_specs=[pl.BlockSpec((_RB, 128), lambda i: (i, 0)),
                   pl.BlockSpec((_RB, 128), lambda i: (i, 0))],
        out_shape=[jax.ShapeDtypeStruct((N, 128), jnp.float32),
                   jax.ShapeDtypeStruct((N, 128), jnp.float32)],
    )(x, W, b2, dinv)


def _asm0_body(pa_ref, pb_ref, ha_ref, hb_ref, d_ref,
               oa_ref, ob_ref, ssum_ref, ssq_ref):
    d = d_ref[...]
    oa = (pa_ref[0] + pa_ref[1] + ha_ref[...]) * d
    ob = (pb_ref[0] + pb_ref[1] + hb_ref[...]) * d
    oa_ref[...] = oa
    ob_ref[...] = ob

    @pl.when(pl.program_id(0) == 0)
    def _():
        ssum_ref[...] = jnp.zeros_like(ssum_ref)
        ssq_ref[...] = jnp.zeros_like(ssq_ref)
    ssum_ref[:, :128] += jnp.sum(oa, 0, keepdims=True)
    ssum_ref[:, 128:] += jnp.sum(ob, 0, keepdims=True)
    ssq_ref[:, :128] += jnp.sum(oa * oa, 0, keepdims=True)
    ssq_ref[:, 128:] += jnp.sum(ob * ob, 0, keepdims=True)


def _asm0_call(pa, pb, ha, hb, dinv):
    return pl.pallas_call(
        _asm0_body,
        grid=(N // _RB,),
        in_specs=[
            pl.BlockSpec((NC, _RB, 128), lambda i: (0, i, 0)),
            pl.BlockSpec((NC, _RB, 128), lambda i: (0, i, 0)),
            pl.BlockSpec((_RB, 128), lambda i: (i, 0)),
            pl.BlockSpec((_RB, 128), lambda i: (i, 0)),
            pl.BlockSpec((_RB, 1), lambda i: (i, 0)),
        ],
        out_specs=[pl.BlockSpec((_RB, 128), lambda i: (i, 0)),
                   pl.BlockSpec((_RB, 128), lambda i: (i, 0)),
                   pl.BlockSpec((1, 256), lambda i: (0, 0)),
                   pl.BlockSpec((1, 256), lambda i: (0, 0))],
        out_shape=[jax.ShapeDtypeStruct((N, 128), jnp.float32),
                   jax.ShapeDtypeStruct((N, 128), jnp.float32),
                   jax.ShapeDtypeStruct((1, 256), jnp.float32),
                   jax.ShapeDtypeStruct((1, 256), jnp.float32)],
    )(pa, pb, ha, hb, dinv)


def _bnmm_body(oa_ref, ob_ref, ssum_ref, ssq_ref, g_ref, be_ref,
               w_ref, b_ref, d_ref, *out_refs):
    mu = ssum_ref[...] / N
    var = ssq_ref[...] / N - mu * mu
    sc = g_ref[...] * lax.rsqrt(var + 1e-5)
    t = be_ref[...] - mu * sc
    z = jnp.concatenate([oa_ref[...], ob_ref[...]], axis=1)
    a = jnp.maximum(z * sc + t, 0.0)
    h = (jnp.dot(a, w_ref[...], preferred_element_type=jnp.float32)
         + b_ref[...]) * d_ref[...]
    for k, o_ref in enumerate(out_refs):
        o_ref[...] = h[:, k * 128:(k + 1) * 128]


def _bnmm_call(oa, ob, ssum, ssq, g2, be2, W, b2, dinv):
    d_in, d_out = W.shape
    n_out = d_out // 128
    return pl.pallas_call(
        _bnmm_body,
        grid=(N // _RB,),
        in_specs=[
            pl.BlockSpec((_RB, 128), lambda i: (i, 0)),
            pl.BlockSpec((_RB, 128), lambda i: (i, 0)),
            pl.BlockSpec((1, d_in), lambda i: (0, 0)),
            pl.BlockSpec((1, d_in), lambda i: (0, 0)),
            pl.BlockSpec((1, d_in), lambda i: (0, 0)),
            pl.BlockSpec((1, d_in), lambda i: (0, 0)),
            pl.BlockSpec(W.shape, lambda i: (0, 0)),
            pl.BlockSpec((1, d_out), lambda i: (0, 0)),
            pl.BlockSpec((_RB, 1), lambda i: (i, 0)),
        ],
        out_specs=[pl.BlockSpec((_RB, 128), lambda i: (i, 0))
                   for _ in range(n_out)],
        out_shape=[jax.ShapeDtypeStruct((N, 128), jnp.float32)
                   for _ in range(n_out)],
    )(oa, ob, ssum, ssq, g2, be2, W, b2, dinv)


def _asm1_body(q0, q1, q2, q3, h0, h1, h2, h3, d_ref,
               o_ref, ssum_ref, ssq_ref):
    d = d_ref[...]

    @pl.when(pl.program_id(0) == 0)
    def _():
        ssum_ref[...] = jnp.zeros_like(ssum_ref)
        ssq_ref[...] = jnp.zeros_like(ssq_ref)
    for k, (q, hh) in enumerate(zip((q0, q1, q2, q3), (h0, h1, h2, h3))):
        o = (q[0] + q[1] + hh[...]) * d
        o_ref[:, k * 128:(k + 1) * 128] = o
        ssum_ref[:, k * 128:(k + 1) * 128] += jnp.sum(o, 0, keepdims=True)
        ssq_ref[:, k * 128:(k + 1) * 128] += jnp.sum(o * o, 0, keepdims=True)


def _asm1_call(qs, hs, dinv):
    return pl.pallas_call(
        _asm1_body,
        grid=(N // _RB,),
        in_specs=[pl.BlockSpec((NC, _RB, 128), lambda i: (0, i, 0))] * 4
        + [pl.BlockSpec((_RB, 128), lambda i: (i, 0))] * 4
        + [pl.BlockSpec((_RB, 1), lambda i: (i, 0))],
        out_specs=[pl.BlockSpec((_RB, 512), lambda i: (i, 0)),
                   pl.BlockSpec((1, 512), lambda i: (0, 0)),
                   pl.BlockSpec((1, 512), lambda i: (0, 0))],
        out_shape=[jax.ShapeDtypeStruct((N, 512), jnp.float32),
                   jax.ShapeDtypeStruct((1, 512), jnp.float32),
                   jax.ShapeDtypeStruct((1, 512), jnp.float32)],
    )(*qs, *hs, dinv)


def _pool_body(o_ref, ssum_ref, ssq_ref, g_ref, be_ref, batch_ref,
               psum_ref, cnt_ref):
    mu = ssum_ref[...] / N
    var = ssq_ref[...] / N - mu * mu
    sc = g_ref[...] * lax.rsqrt(var + 1e-5)
    t = be_ref[...] - mu * sc
    z = jnp.maximum(o_ref[...] * sc + t, 0.0)
    ids = lax.broadcasted_iota(jnp.int32, (_RB, G), 1)
    m = (ids == jnp.broadcast_to(batch_ref[...], (_RB, G))).astype(jnp.float32)

    @pl.when(pl.program_id(0) == 0)
    def _():
        psum_ref[...] = jnp.zeros_like(psum_ref)
        cnt_ref[...] = jnp.zeros_like(cnt_ref)
    dn = (((0,), (0,)), ((), ()))
    psum_ref[...] += lax.dot_general(m, z, dn,
                                     preferred_element_type=jnp.float32)
    cnt_ref[...] += lax.dot_general(m, jnp.ones((_RB, 128), jnp.float32), dn,
                                    preferred_element_type=jnp.float32)


def _pool_call(out1, ssum, ssq, g2, be2, batch_col):
    return pl.pallas_call(
        _pool_body,
        grid=(N // _RB,),
        in_specs=[
            pl.BlockSpec((_RB, 512), lambda i: (i, 0)),
            pl.BlockSpec((1, 512), lambda i: (0, 0)),
            pl.BlockSpec((1, 512), lambda i: (0, 0)),
            pl.BlockSpec((1, 512), lambda i: (0, 0)),
            pl.BlockSpec((1, 512), lambda i: (0, 0)),
            pl.BlockSpec((_RB, 1), lambda i: (i, 0)),
        ],
        out_specs=[pl.BlockSpec((G, 512), lambda i: (0, 0)),
                   pl.BlockSpec((G, 128), lambda i: (0, 0))],
        out_shape=[jax.ShapeDtypeStruct((G, 512), jnp.float32),
                   jax.ShapeDtypeStruct((G, 128), jnp.float32)],
    )(out1, ssum, ssq, g2, be2, batch_col)


def _head_body(ps_ref, cnt_ref, w2_ref, b2_ref, w1_ref, b1_ref,
               w0_ref, b0_ref, o_ref):
    cnt = jnp.maximum(cnt_ref[:, 0:1], 1.0)
    h = ps_ref[...] / cnt
    h = jnp.maximum(jnp.dot(h, w2_ref[...], preferred_element_type=jnp.float32)
                    + b2_ref[...], 0.0)
    h = jnp.maximum(jnp.dot(h, w1_ref[...], preferred_element_type=jnp.float32)
                    + b1_ref[...], 0.0)
    h = jnp.maximum(jnp.dot(h, w0_ref[...], preferred_element_type=jnp.float32)
                    + b0_ref[...], 0.0)
    o_ref[...] = h


def _head_call(psum, cnt, Wl2, bl2, Wl1, bl1, Wl0, bl0):
    return pl.pallas_call(
        _head_body,
        out_shape=jax.ShapeDtypeStruct((G, Wl0.shape[1]), jnp.float32),
    )(psum, cnt, Wl2, bl2.reshape(1, -1), Wl1, bl1.reshape(1, -1),
      Wl0, bl0.reshape(1, -1))


def kernel(x, edge_index, edge_weight, batch, Wc0, bc0, g0, be0,
           Wc1, bc1, g1, be1, Wl2, bl2, Wl1, bl1, Wl0, bl0):
    src1 = edge_index[0]
    dst1 = edge_index[1]

    wflat = edge_weight
    deg2 = _agg_call(jnp.ones((N, 128), jnp.float32), src1, dst1, wflat)
    dinv = _dinv_call(deg2)

    h0a, h0b = _mm0_call(x, Wc0, bc0.reshape(1, -1), dinv)
    pa = _agg_call(h0a, src1, dst1, wflat)
    pb = _agg_call(h0b, src1, dst1, wflat)
    oa, ob, ssum0, ssq0 = _asm0_call(pa, pb, h0a, h0b, dinv)
    h1s = _bnmm_call(oa, ob, ssum0, ssq0, g0.reshape(1, -1),
                     be0.reshape(1, -1), Wc1, bc1.reshape(1, -1), dinv)

    qs = [_agg_call(h1s[k], src1, dst1, wflat) for k in range(4)]
    out1, ssum1, ssq1 = _asm1_call(qs, h1s, dinv)

    psum, cnt = _pool_call(out1, ssum1, ssq1, g1.reshape(1, -1),
                           be1.reshape(1, -1), batch.reshape(N, 1))
    return _head_call(psum, cnt, Wl2, bl2, Wl1, bl1, Wl0, bl0)


# speedup vs baseline: 10.9047x; 1.2680x over previous
import jax
import jax.numpy as jnp
from jax import lax
from jax.experimental import pallas as pl
from jax.experimental.pallas import tpu as pltpu
from jax.experimental.pallas import tpu_sc as plsc

N = 10000
E = 320000
G = 16
NC = 2
NS = 16
NW = NC * NS
EW = E // NW
BQ = 80
QI = 5
BI = BQ * QI
ITERS = EW // BI
NPAD = 10240
NT = NPAD // NS
NZ = NT // 5


def _sc_mesh():
    return plsc.VectorSubcoreMesh(core_axis_name="c", subcore_axis_name="s",
                                  num_cores=NC, num_subcores=NS)


def _agg_body(h_h, src1, dst1, wflat, out, acc, stage, sidx, wv,
              rows0, rows1, rows2, gs0, gs1, gs2, ss0, ss1, ss2, psem, didx):
    c = lax.axis_index("c")
    s = lax.axis_index("s")
    wid = c * NS + s
    zf = jnp.zeros((16,), jnp.float32)
    rows = (rows0, rows1, rows2)
    gsem = (gs0, gs1, gs2)
    ssem = (ss0, ss1, ss2)

    def zloop(r, _):
        for k in range(8):
            stage.at[r][pl.ds(k * 16, 16)] = zf
        return 0
    lax.fori_loop(0, NZ, zloop, 0)
    for k in range(5):
        pltpu.sync_copy(stage, acc.at[pl.ds(s * NT + k * NZ, NZ)])
    plsc.subcore_barrier()

    def body(it, _):
        base = wid * EW + it * BI
        pf = [pltpu.async_copy(src1.at[pl.ds(base, BI)], sidx, psem),
              pltpu.async_copy(wflat.at[pl.ds(base, BI)], wv, psem),
              pltpu.async_copy(dst1.at[pl.ds(base, BI)], didx, psem)]
        for d in pf:
            d.wait()
        gathers = [None] * QI
        scatters = [None] * QI

        def _gather(qq):
            return pltpu.async_copy(
                h_h.at[sidx.at[pl.ds(qq * BQ, BQ)]],
                rows[qq % 3], gsem[qq % 3])

        gathers[0] = _gather(0)
        gathers[1] = _gather(1)
        for q in range(QI):
            cur = q % 3
            if q + 2 < QI:
                if q >= 1:
                    scatters[q - 1].wait()
                gathers[q + 2] = _gather(q + 2)
            gathers[q].wait()

            def scale(g, _):
                wpk = wv[pl.ds(q * BQ + g * 16, 16)]
                for l in range(16):
                    cfb = lax.gather(
                        wpk, jnp.full((16, 1), l, jnp.int32),
                        lax.GatherDimensionNumbers(
                            offset_dims=(), collapsed_slice_dims=(0,),
                            start_index_map=(0,)),
                        (1,), mode=lax.GatherScatterMode.PROMISE_IN_BOUNDS)
                    rr = rows[cur].at[g * 16 + l]
                    for k in range(8):
                        sl = pl.ds(k * 16, 16)
                        rr[sl] = rr[sl] * cfb
                return 0
            lax.fori_loop(0, BQ // 16, scale, 0)
            scatters[q] = pltpu.async_copy(
                rows[cur], acc.at[didx.at[pl.ds(q * BQ, BQ)]],
                ssem[cur], add=True)
        scatters[QI - 2].wait()
        scatters[QI - 1].wait()
        return 0
    lax.fori_loop(0, ITERS, body, 0)
    plsc.subcore_barrier()
    for k in range(5):
        pltpu.sync_copy(acc.at[pl.ds(s * NT + k * NZ, NZ)], stage)
        pltpu.sync_copy(stage, out.at[c, pl.ds(s * NT + k * NZ, NZ)])


def _agg_call(h_chunk, src1, dst1, wflat):
    return pl.kernel(
        _agg_body,
        out_type=jax.ShapeDtypeStruct((NC, NPAD, 128), jnp.float32),
        mesh=_sc_mesh(),
        scratch_types=[
            pltpu.VMEM_SHARED((NPAD, 128), jnp.float32),
            pltpu.VMEM((NZ, 128), jnp.float32),
            pltpu.VMEM((BI,), jnp.int32),
            pltpu.VMEM((BI,), jnp.float32),
            pltpu.VMEM((BQ, 128), jnp.float32),
            pltpu.VMEM((BQ, 128), jnp.float32),
            pltpu.VMEM((BQ, 128), jnp.float32),
            pltpu.SemaphoreType.DMA,
            pltpu.SemaphoreType.DMA,
            pltpu.SemaphoreType.DMA,
            pltpu.SemaphoreType.DMA,
            pltpu.SemaphoreType.DMA,
            pltpu.SemaphoreType.DMA,
            pltpu.SemaphoreType.DMA,
            pltpu.VMEM((BI,), jnp.int32),
        ],
    )(h_chunk, src1, dst1, wflat)


_RB = 1000


def _dinv_body(deg2_ref, dinv_ref):
    d = deg2_ref[0][:, 0:1] + deg2_ref[1][:, 0:1] + 1.0
    dinv_ref[...] = jnp.where(d > 0, lax.rsqrt(jnp.maximum(d, 1e-12)), 0.0)


def _dinv_call(deg2):
    return pl.pallas_call(
        _dinv_body,
        grid=(1,),
        in_specs=[pl.BlockSpec((NC, N, 128), lambda i: (0, 0, 0))],
        out_specs=pl.BlockSpec((N, 1), lambda i: (0, 0)),
        out_shape=jax.ShapeDtypeStruct((N, 1), jnp.float32),
    )(deg2)


def _mm0_body(x_ref, w_ref, b_ref, d_ref, oa_ref, ob_ref):
    h = (jnp.dot(x_ref[...], w_ref[...], preferred_element_type=jnp.float32)
         + b_ref[...]) * d_ref[...]
    oa_ref[...] = h[:, :128]
    ob_ref[...] = h[:, 128:]


def _mm0_call(x, W, b2, dinv):
    d_out = W.shape[1]
    return pl.pallas_call(
        _mm0_body,
        grid=(N // _RB,),
        in_specs=[
            pl.BlockSpec((_RB, x.shape[1]), lambda i: (i, 0)),
            pl.BlockSpec(W.shape, lambda i: (0, 0)),
            pl.BlockSpec((1, d_out), lambda i: (0, 0)),
            pl.BlockSpec((_RB, 1), lambda i: (i, 0)),
        ],
        out_specs=[pl.BlockSpec((_RB, 128), lambda i: (i, 0)),
                   pl.BlockSpec((_RB, 128), lambda i: (i, 0))],
        out_shape=[jax.ShapeDtypeStruct((N, 128), jnp.float32),
                   jax.ShapeDtypeStruct((N, 128), jnp.float32)],
    )(x, W, b2, dinv)


def _asm0_body(pa_ref, pb_ref, ha_ref, hb_ref, d_ref,
               oa_ref, ob_ref, ssum_ref, ssq_ref):
    d = d_ref[...]
    oa = (pa_ref[0] + pa_ref[1] + ha_ref[...]) * d
    ob = (pb_ref[0] + pb_ref[1] + hb_ref[...]) * d
    oa_ref[...] = oa
    ob_ref[...] = ob

    @pl.when(pl.program_id(0) == 0)
    def _():
        ssum_ref[...] = jnp.zeros_like(ssum_ref)
        ssq_ref[...] = jnp.zeros_like(ssq_ref)
    ssum_ref[:, :128] += jnp.sum(oa, 0, keepdims=True)
    ssum_ref[:, 128:] += jnp.sum(ob, 0, keepdims=True)
    ssq_ref[:, :128] += jnp.sum(oa * oa, 0, keepdims=True)
    ssq_ref[:, 128:] += jnp.sum(ob * ob, 0, keepdims=True)


def _asm0_call(pa, pb, ha, hb, dinv):
    return pl.pallas_call(
        _asm0_body,
        grid=(N // _RB,),
        in_specs=[
            pl.BlockSpec((NC, _RB, 128), lambda i: (0, i, 0)),
            pl.BlockSpec((NC, _RB, 128), lambda i: (0, i, 0)),
            pl.BlockSpec((_RB, 128), lambda i: (i, 0)),
            pl.BlockSpec((_RB, 128), lambda i: (i, 0)),
            pl.BlockSpec((_RB, 1), lambda i: (i, 0)),
        ],
        out_specs=[pl.BlockSpec((_RB, 128), lambda i: (i, 0)),
                   pl.BlockSpec((_RB, 128), lambda i: (i, 0)),
                   pl.BlockSpec((1, 256), lambda i: (0, 0)),
                   pl.BlockSpec((1, 256), lambda i: (0, 0))],
        out_shape=[jax.ShapeDtypeStruct((N, 128), jnp.float32),
                   jax.ShapeDtypeStruct((N, 128), jnp.float32),
                   jax.ShapeDtypeStruct((1, 256), jnp.float32),
                   jax.ShapeDtypeStruct((1, 256), jnp.float32)],
    )(pa, pb, ha, hb, dinv)


def _bnmm_body(oa_ref, ob_ref, ssum_ref, ssq_ref, g_ref, be_ref,
               w_ref, b_ref, d_ref, *out_refs):
    mu = ssum_ref[...] / N
    var = ssq_ref[...] / N - mu * mu
    sc = g_ref[...] * lax.rsqrt(var + 1e-5)
    t = be_ref[...] - mu * sc
    z = jnp.concatenate([oa_ref[...], ob_ref[...]], axis=1)
    a = jnp.maximum(z * sc + t, 0.0)
    h = (jnp.dot(a, w_ref[...], preferred_element_type=jnp.float32)
         + b_ref[...]) * d_ref[...]
    for k, o_ref in enumerate(out_refs):
        o_ref[...] = h[:, k * 128:(k + 1) * 128]


def _bnmm_call(oa, ob, ssum, ssq, g2, be2, W, b2, dinv):
    d_in, d_out = W.shape
    n_out = d_out // 128
    return pl.pallas_call(
        _bnmm_body,
        grid=(N // _RB,),
        in_specs=[
            pl.BlockSpec((_RB, 128), lambda i: (i, 0)),
            pl.BlockSpec((_RB, 128), lambda i: (i, 0)),
            pl.BlockSpec((1, d_in), lambda i: (0, 0)),
            pl.BlockSpec((1, d_in), lambda i: (0, 0)),
            pl.BlockSpec((1, d_in), lambda i: (0, 0)),
            pl.BlockSpec((1, d_in), lambda i: (0, 0)),
            pl.BlockSpec(W.shape, lambda i: (0, 0)),
            pl.BlockSpec((1, d_out), lambda i: (0, 0)),
            pl.BlockSpec((_RB, 1), lambda i: (i, 0)),
        ],
        out_specs=[pl.BlockSpec((_RB, 128), lambda i: (i, 0))
                   for _ in range(n_out)],
        out_shape=[jax.ShapeDtypeStruct((N, 128), jnp.float32)
                   for _ in range(n_out)],
    )(oa, ob, ssum, ssq, g2, be2, W, b2, dinv)


def _asm1_body(q0, q1, q2, q3, h0, h1, h2, h3, d_ref,
               o_ref, ssum_ref, ssq_ref):
    d = d_ref[...]

    @pl.when(pl.program_id(0) == 0)
    def _():
        ssum_ref[...] = jnp.zeros_like(ssum_ref)
        ssq_ref[...] = jnp.zeros_like(ssq_ref)
    for k, (q, hh) in enumerate(zip((q0, q1, q2, q3), (h0, h1, h2, h3))):
        o = (q[0] + q[1] + hh[...]) * d
        o_ref[:, k * 128:(k + 1) * 128] = o
        ssum_ref[:, k * 128:(k + 1) * 128] += jnp.sum(o, 0, keepdims=True)
        ssq_ref[:, k * 128:(k + 1) * 128] += jnp.sum(o * o, 0, keepdims=True)


def _asm1_call(qs, hs, dinv):
    return pl.pallas_call(
        _asm1_body,
        grid=(N // _RB,),
        in_specs=[pl.BlockSpec((NC, _RB, 128), lambda i: (0, i, 0))] * 4
        + [pl.BlockSpec((_RB, 128), lambda i: (i, 0))] * 4
        + [pl.BlockSpec((_RB, 1), lambda i: (i, 0))],
        out_specs=[pl.BlockSpec((_RB, 512), lambda i: (i, 0)),
                   pl.BlockSpec((1, 512), lambda i: (0, 0)),
                   pl.BlockSpec((1, 512), lambda i: (0, 0))],
        out_shape=[jax.ShapeDtypeStruct((N, 512), jnp.float32),
                   jax.ShapeDtypeStruct((1, 512), jnp.float32),
                   jax.ShapeDtypeStruct((1, 512), jnp.float32)],
    )(*qs, *hs, dinv)


def _pool_body(o_ref, ssum_ref, ssq_ref, g_ref, be_ref, batch_ref,
               psum_ref, cnt_ref):
    mu = ssum_ref[...] / N
    var = ssq_ref[...] / N - mu * mu
    sc = g_ref[...] * lax.rsqrt(var + 1e-5)
    t = be_ref[...] - mu * sc
    z = jnp.maximum(o_ref[...] * sc + t, 0.0)
    ids = lax.broadcasted_iota(jnp.int32, (_RB, G), 1)
    m = (ids == jnp.broadcast_to(batch_ref[...], (_RB, G))).astype(jnp.float32)

    @pl.when(pl.program_id(0) == 0)
    def _():
        psum_ref[...] = jnp.zeros_like(psum_ref)
        cnt_ref[...] = jnp.zeros_like(cnt_ref)
    dn = (((0,), (0,)), ((), ()))
    psum_ref[...] += lax.dot_general(m, z, dn,
                                     preferred_element_type=jnp.float32)
    cnt_ref[...] += lax.dot_general(m, jnp.ones((_RB, 128), jnp.float32), dn,
                                    preferred_element_type=jnp.float32)


def _pool_call(out1, ssum, ssq, g2, be2, batch_col):
    return pl.pallas_call(
        _pool_body,
        grid=(N // _RB,),
        in_specs=[
            pl.BlockSpec((_RB, 512), lambda i: (i, 0)),
            pl.BlockSpec((1, 512), lambda i: (0, 0)),
            pl.BlockSpec((1, 512), lambda i: (0, 0)),
            pl.BlockSpec((1, 512), lambda i: (0, 0)),
            pl.BlockSpec((1, 512), lambda i: (0, 0)),
            pl.BlockSpec((_RB, 1), lambda i: (i, 0)),
        ],
        out_specs=[pl.BlockSpec((G, 512), lambda i: (0, 0)),
                   pl.BlockSpec((G, 128), lambda i: (0, 0))],
        out_shape=[jax.ShapeDtypeStruct((G, 512), jnp.float32),
                   jax.ShapeDtypeStruct((G, 128), jnp.float32)],
    )(out1, ssum, ssq, g2, be2, batch_col)


def _head_body(ps_ref, cnt_ref, w2_ref, b2_ref, w1_ref, b1_ref,
               w0_ref, b0_ref, o_ref):
    cnt = jnp.maximum(cnt_ref[:, 0:1], 1.0)
    h = ps_ref[...] / cnt
    h = jnp.maximum(jnp.dot(h, w2_ref[...], preferred_element_type=jnp.float32)
                    + b2_ref[...], 0.0)
    h = jnp.maximum(jnp.dot(h, w1_ref[...], preferred_element_type=jnp.float32)
                    + b1_ref[...], 0.0)
    h = jnp.maximum(jnp.dot(h, w0_ref[...], preferred_element_type=jnp.float32)
                    + b0_ref[...], 0.0)
    o_ref[...] = h


def _head_call(psum, cnt, Wl2, bl2, Wl1, bl1, Wl0, bl0):
    return pl.pallas_call(
        _head_body,
        out_shape=jax.ShapeDtypeStruct((G, Wl0.shape[1]), jnp.float32),
    )(psum, cnt, Wl2, bl2.reshape(1, -1), Wl1, bl1.reshape(1, -1),
      Wl0, bl0.reshape(1, -1))


def kernel(x, edge_index, edge_weight, batch, Wc0, bc0, g0, be0,
           Wc1, bc1, g1, be1, Wl2, bl2, Wl1, bl1, Wl0, bl0):
    src1 = edge_index[0]
    dst1 = edge_index[1]

    wflat = edge_weight
    deg2 = _agg_call(jnp.ones((N, 128), jnp.float32), src1, dst1, wflat)
    dinv = _dinv_call(deg2)

    h0a, h0b = _mm0_call(x, Wc0, bc0.reshape(1, -1), dinv)
    pa = _agg_call(h0a, src1, dst1, wflat)
    pb = _agg_call(h0b, src1, dst1, wflat)
    oa, ob, ssum0, ssq0 = _asm0_call(pa, pb, h0a, h0b, dinv)
    h1s = _bnmm_call(oa, ob, ssum0, ssq0, g0.reshape(1, -1),
                     be0.reshape(1, -1), Wc1, bc1.reshape(1, -1), dinv)

    qs = [_agg_call(h1s[k], src1, dst1, wflat) for k in range(4)]
    out1, ssum1, ssq1 = _asm1_call(qs, h1s, dinv)

    psum, cnt = _pool_call(out1, ssum1, ssq1, g1.reshape(1, -1),
                           be1.reshape(1, -1), batch.reshape(N, 1))
    return _head_call(psum, cnt, Wl2, bl2, Wl1, bl1, Wl0, bl0)


# speedup vs baseline: 10.9453x; 1.0037x over previous
import jax
import jax.numpy as jnp
from jax import lax
from jax.experimental import pallas as pl
from jax.experimental.pallas import tpu as pltpu
from jax.experimental.pallas import tpu_sc as plsc

N = 10000
E = 320000
G = 16
NC = 2
NS = 16
NW = NC * NS
EW = E // NW
BQ = 80
QI = 5
BI = BQ * QI
ITERS = EW // BI
NPAD = 10240
NT = NPAD // NS
NZ = NT // 5


def _sc_mesh():
    return plsc.VectorSubcoreMesh(core_axis_name="c", subcore_axis_name="s",
                                  num_cores=NC, num_subcores=NS)


def _agg_body(h_h, src1, dst1, wflat, out, acc, stage, sidx, wv,
              rows0, rows1, rows2, gs0, gs1, gs2, ss0, ss1, ss2, psem, didx):
    c = lax.axis_index("c")
    s = lax.axis_index("s")
    wid = c * NS + s
    zf = jnp.zeros((16,), jnp.float32)
    rows = (rows0, rows1, rows2)
    gsem = (gs0, gs1, gs2)
    ssem = (ss0, ss1, ss2)

    def zloop(r, _):
        for k in range(8):
            stage.at[r][pl.ds(k * 16, 16)] = zf
        return 0
    lax.fori_loop(0, NZ, zloop, 0)
    for k in range(5):
        pltpu.sync_copy(stage, acc.at[pl.ds(s * NT + k * NZ, NZ)])
    plsc.subcore_barrier()

    def body(it, _):
        base = wid * EW + it * BI
        pf = [pltpu.async_copy(src1.at[pl.ds(base, BI)], sidx, psem),
              pltpu.async_copy(wflat.at[pl.ds(base, BI)], wv, psem),
              pltpu.async_copy(dst1.at[pl.ds(base, BI)], didx, psem)]
        for d in pf:
            d.wait()
        gathers = [None] * QI
        scatters = [None] * QI

        def _gather(qq):
            return pltpu.async_copy(
                h_h.at[sidx.at[pl.ds(qq * BQ, BQ)]],
                rows[qq % 3], gsem[qq % 3])

        gathers[0] = _gather(0)
        gathers[1] = _gather(1)
        for q in range(QI):
            cur = q % 3
            if q + 2 < QI:
                if q >= 1:
                    scatters[q - 1].wait()
                gathers[q + 2] = _gather(q + 2)
            gathers[q].wait()

            def scale(g, _):
                wpk = wv[pl.ds(q * BQ + g * 16, 16)]
                for l in range(16):
                    cfb = lax.gather(
                        wpk, jnp.full((16, 1), l, jnp.int32),
                        lax.GatherDimensionNumbers(
                            offset_dims=(), collapsed_slice_dims=(0,),
                            start_index_map=(0,)),
                        (1,), mode=lax.GatherScatterMode.PROMISE_IN_BOUNDS)
                    rr = rows[cur].at[g * 16 + l]
                    for k in range(8):
                        sl = pl.ds(k * 16, 16)
                        rr[sl] = rr[sl] * cfb
                return 0
            lax.fori_loop(0, BQ // 16, scale, 0)
            scatters[q] = pltpu.async_copy(
                rows[cur], acc.at[didx.at[pl.ds(q * BQ, BQ)]],
                ssem[cur], add=True)
        scatters[QI - 3].wait()
        scatters[QI - 2].wait()
        scatters[QI - 1].wait()
        return 0
    lax.fori_loop(0, ITERS, body, 0)
    plsc.subcore_barrier()
    for k in range(5):
        pltpu.sync_copy(acc.at[pl.ds(s * NT + k * NZ, NZ)], stage)
        pltpu.sync_copy(stage, out.at[c, pl.ds(s * NT + k * NZ, NZ)])


def _agg_call(h_chunk, src1, dst1, wflat):
    return pl.kernel(
        _agg_body,
        out_type=jax.ShapeDtypeStruct((NC, NPAD, 128), jnp.float32),
        mesh=_sc_mesh(),
        scratch_types=[
            pltpu.VMEM_SHARED((NPAD, 128), jnp.float32),
            pltpu.VMEM((NZ, 128), jnp.float32),
            pltpu.VMEM((BI,), jnp.int32),
            pltpu.VMEM((BI,), jnp.float32),
            pltpu.VMEM((BQ, 128), jnp.float32),
            pltpu.VMEM((BQ, 128), jnp.float32),
            pltpu.VMEM((BQ, 128), jnp.float32),
            pltpu.SemaphoreType.DMA,
            pltpu.SemaphoreType.DMA,
            pltpu.SemaphoreType.DMA,
            pltpu.SemaphoreType.DMA,
            pltpu.SemaphoreType.DMA,
            pltpu.SemaphoreType.DMA,
            pltpu.SemaphoreType.DMA,
            pltpu.VMEM((BI,), jnp.int32),
        ],
    )(h_chunk, src1, dst1, wflat)


_RB = 1000


def _dinv_body(deg2_ref, dinv_ref):
    d = deg2_ref[0][:, 0:1] + deg2_ref[1][:, 0:1] + 1.0
    dinv_ref[...] = jnp.where(d > 0, lax.rsqrt(jnp.maximum(d, 1e-12)), 0.0)


def _dinv_call(deg2):
    return pl.pallas_call(
        _dinv_body,
        grid=(1,),
        in_specs=[pl.BlockSpec((NC, N, 128), lambda i: (0, 0, 0))],
        out_specs=pl.BlockSpec((N, 1), lambda i: (0, 0)),
        out_shape=jax.ShapeDtypeStruct((N, 1), jnp.float32),
    )(deg2)


def _mm0_body(x_ref, w_ref, b_ref, d_ref, oa_ref, ob_ref):
    h = (jnp.dot(x_ref[...], w_ref[...], preferred_element_type=jnp.float32)
         + b_ref[...]) * d_ref[...]
    oa_ref[...] = h[:, :128]
    ob_ref[...] = h[:, 128:]


def _mm0_call(x, W, b2, dinv):
    d_out = W.shape[1]
    return pl.pallas_call(
        _mm0_body,
        grid=(N // _RB,),
        in_specs=[
            pl.BlockSpec((_RB, x.shape[1]), lambda i: (i, 0)),
            pl.BlockSpec(W.shape, lambda i: (0, 0)),
            pl.BlockSpec((1, d_out), lambda i: (0, 0)),
            pl.BlockSpec((_RB, 1), lambda i: (i, 0)),
        ],
        out_specs=[pl.BlockSpec((_RB, 128), lambda i: (i, 0)),
                   pl.BlockSpec((_RB, 128), lambda i: (i, 0))],
        out_shape=[jax.ShapeDtypeStruct((N, 128), jnp.float32),
                   jax.ShapeDtypeStruct((N, 128), jnp.float32)],
    )(x, W, b2, dinv)


def _asm0_body(pa_ref, pb_ref, ha_ref, hb_ref, d_ref,
               oa_ref, ob_ref, ssum_ref, ssq_ref):
    d = d_ref[...]
    oa = (pa_ref[0] + pa_ref[1] + ha_ref[...]) * d
    ob = (pb_ref[0] + pb_ref[1] + hb_ref[...]) * d
    oa_ref[...] = oa
    ob_ref[...] = ob

    @pl.when(pl.program_id(0) == 0)
    def _():
        ssum_ref[...] = jnp.zeros_like(ssum_ref)
        ssq_ref[...] = jnp.zeros_like(ssq_ref)
    ssum_ref[:, :128] += jnp.sum(oa, 0, keepdims=True)
    ssum_ref[:, 128:] += jnp.sum(ob, 0, keepdims=True)
    ssq_ref[:, :128] += jnp.sum(oa * oa, 0, keepdims=True)
    ssq_ref[:, 128:] += jnp.sum(ob * ob, 0, keepdims=True)


def _asm0_call(pa, pb, ha, hb, dinv):
    return pl.pallas_call(
        _asm0_body,
        grid=(N // _RB,),
        in_specs=[
            pl.BlockSpec((NC, _RB, 128), lambda i: (0, i, 0)),
            pl.BlockSpec((NC, _RB, 128), lambda i: (0, i, 0)),
            pl.BlockSpec((_RB, 128), lambda i: (i, 0)),
            pl.BlockSpec((_RB, 128), lambda i: (i, 0)),
            pl.BlockSpec((_RB, 1), lambda i: (i, 0)),
        ],
        out_specs=[pl.BlockSpec((_RB, 128), lambda i: (i, 0)),
                   pl.BlockSpec((_RB, 128), lambda i: (i, 0)),
                   pl.BlockSpec((1, 256), lambda i: (0, 0)),
                   pl.BlockSpec((1, 256), lambda i: (0, 0))],
        out_shape=[jax.ShapeDtypeStruct((N, 128), jnp.float32),
                   jax.ShapeDtypeStruct((N, 128), jnp.float32),
                   jax.ShapeDtypeStruct((1, 256), jnp.float32),
                   jax.ShapeDtypeStruct((1, 256), jnp.float32)],
    )(pa, pb, ha, hb, dinv)


def _bnmm_body(oa_ref, ob_ref, ssum_ref, ssq_ref, g_ref, be_ref,
               w_ref, b_ref, d_ref, *out_refs):
    mu = ssum_ref[...] / N
    var = ssq_ref[...] / N - mu * mu
    sc = g_ref[...] * lax.rsqrt(var + 1e-5)
    t = be_ref[...] - mu * sc
    z = jnp.concatenate([oa_ref[...], ob_ref[...]], axis=1)
    a = jnp.maximum(z * sc + t, 0.0)
    h = (jnp.dot(a, w_ref[...], preferred_element_type=jnp.float32)
         + b_ref[...]) * d_ref[...]
    for k, o_ref in enumerate(out_refs):
        o_ref[...] = h[:, k * 128:(k + 1) * 128]


def _bnmm_call(oa, ob, ssum, ssq, g2, be2, W, b2, dinv):
    d_in, d_out = W.shape
    n_out = d_out // 128
    return pl.pallas_call(
        _bnmm_body,
        grid=(N // _RB,),
        in_specs=[
            pl.BlockSpec((_RB, 128), lambda i: (i, 0)),
            pl.BlockSpec((_RB, 128), lambda i: (i, 0)),
            pl.BlockSpec((1, d_in), lambda i: (0, 0)),
            pl.BlockSpec((1, d_in), lambda i: (0, 0)),
            pl.BlockSpec((1, d_in), lambda i: (0, 0)),
            pl.BlockSpec((1, d_in), lambda i: (0, 0)),
            pl.BlockSpec(W.shape, lambda i: (0, 0)),
            pl.BlockSpec((1, d_out), lambda i: (0, 0)),
            pl.BlockSpec((_RB, 1), lambda i: (i, 0)),
        ],
        out_specs=[pl.BlockSpec((_RB, 128), lambda i: (i, 0))
                   for _ in range(n_out)],
        out_shape=[jax.ShapeDtypeStruct((N, 128), jnp.float32)
                   for _ in range(n_out)],
    )(oa, ob, ssum, ssq, g2, be2, W, b2, dinv)


def _asm1_body(q0, q1, q2, q3, h0, h1, h2, h3, d_ref,
               o_ref, ssum_ref, ssq_ref):
    d = d_ref[...]

    @pl.when(pl.program_id(0) == 0)
    def _():
        ssum_ref[...] = jnp.zeros_like(ssum_ref)
        ssq_ref[...] = jnp.zeros_like(ssq_ref)
    for k, (q, hh) in enumerate(zip((q0, q1, q2, q3), (h0, h1, h2, h3))):
        o = (q[0] + q[1] + hh[...]) * d
        o_ref[:, k * 128:(k + 1) * 128] = o
        ssum_ref[:, k * 128:(k + 1) * 128] += jnp.sum(o, 0, keepdims=True)
        ssq_ref[:, k * 128:(k + 1) * 128] += jnp.sum(o * o, 0, keepdims=True)


def _asm1_call(qs, hs, dinv):
    return pl.pallas_call(
        _asm1_body,
        grid=(N // _RB,),
        in_specs=[pl.BlockSpec((NC, _RB, 128), lambda i: (0, i, 0))] * 4
        + [pl.BlockSpec((_RB, 128), lambda i: (i, 0))] * 4
        + [pl.BlockSpec((_RB, 1), lambda i: (i, 0))],
        out_specs=[pl.BlockSpec((_RB, 512), lambda i: (i, 0)),
                   pl.BlockSpec((1, 512), lambda i: (0, 0)),
                   pl.BlockSpec((1, 512), lambda i: (0, 0))],
        out_shape=[jax.ShapeDtypeStruct((N, 512), jnp.float32),
                   jax.ShapeDtypeStruct((1, 512), jnp.float32),
                   jax.ShapeDtypeStruct((1, 512), jnp.float32)],
    )(*qs, *hs, dinv)


def _pool_body(o_ref, ssum_ref, ssq_ref, g_ref, be_ref, batch_ref,
               psum_ref, cnt_ref):
    mu = ssum_ref[...] / N
    var = ssq_ref[...] / N - mu * mu
    sc = g_ref[...] * lax.rsqrt(var + 1e-5)
    t = be_ref[...] - mu * sc
    z = jnp.maximum(o_ref[...] * sc + t, 0.0)
    ids = lax.broadcasted_iota(jnp.int32, (_RB, G), 1)
    m = (ids == jnp.broadcast_to(batch_ref[...], (_RB, G))).astype(jnp.float32)

    @pl.when(pl.program_id(0) == 0)
    def _():
        psum_ref[...] = jnp.zeros_like(psum_ref)
        cnt_ref[...] = jnp.zeros_like(cnt_ref)
    dn = (((0,), (0,)), ((), ()))
    psum_ref[...] += lax.dot_general(m, z, dn,
                                     preferred_element_type=jnp.float32)
    cnt_ref[...] += lax.dot_general(m, jnp.ones((_RB, 128), jnp.float32), dn,
                                    preferred_element_type=jnp.float32)


def _pool_call(out1, ssum, ssq, g2, be2, batch_col):
    return pl.pallas_call(
        _pool_body,
        grid=(N // _RB,),
        in_specs=[
            pl.BlockSpec((_RB, 512), lambda i: (i, 0)),
            pl.BlockSpec((1, 512), lambda i: (0, 0)),
            pl.BlockSpec((1, 512), lambda i: (0, 0)),
            pl.BlockSpec((1, 512), lambda i: (0, 0)),
            pl.BlockSpec((1, 512), lambda i: (0, 0)),
            pl.BlockSpec((_RB, 1), lambda i: (i, 0)),
        ],
        out_specs=[pl.BlockSpec((G, 512), lambda i: (0, 0)),
                   pl.BlockSpec((G, 128), lambda i: (0, 0))],
        out_shape=[jax.ShapeDtypeStruct((G, 512), jnp.float32),
                   jax.ShapeDtypeStruct((G, 128), jnp.float32)],
    )(out1, ssum, ssq, g2, be2, batch_col)


def _head_body(ps_ref, cnt_ref, w2_ref, b2_ref, w1_ref, b1_ref,
               w0_ref, b0_ref, o_ref):
    cnt = jnp.maximum(cnt_ref[:, 0:1], 1.0)
    h = ps_ref[...] / cnt
    h = jnp.maximum(jnp.dot(h, w2_ref[...], preferred_element_type=jnp.float32)
                    + b2_ref[...], 0.0)
    h = jnp.maximum(jnp.dot(h, w1_ref[...], preferred_element_type=jnp.float32)
                    + b1_ref[...], 0.0)
    h = jnp.maximum(jnp.dot(h, w0_ref[...], preferred_element_type=jnp.float32)
                    + b0_ref[...], 0.0)
    o_ref[...] = h


def _head_call(psum, cnt, Wl2, bl2, Wl1, bl1, Wl0, bl0):
    return pl.pallas_call(
        _head_body,
        out_shape=jax.ShapeDtypeStruct((G, Wl0.shape[1]), jnp.float32),
    )(psum, cnt, Wl2, bl2.reshape(1, -1), Wl1, bl1.reshape(1, -1),
      Wl0, bl0.reshape(1, -1))


def kernel(x, edge_index, edge_weight, batch, Wc0, bc0, g0, be0,
           Wc1, bc1, g1, be1, Wl2, bl2, Wl1, bl1, Wl0, bl0):
    src1 = edge_index[0]
    dst1 = edge_index[1]

    wflat = edge_weight
    deg2 = _agg_call(jnp.ones((N, 128), jnp.float32), src1, dst1, wflat)
    dinv = _dinv_call(deg2)

    h0a, h0b = _mm0_call(x, Wc0, bc0.reshape(1, -1), dinv)
    pa = _agg_call(h0a, src1, dst1, wflat)
    pb = _agg_call(h0b, src1, dst1, wflat)
    oa, ob, ssum0, ssq0 = _asm0_call(pa, pb, h0a, h0b, dinv)
    h1s = _bnmm_call(oa, ob, ssum0, ssq0, g0.reshape(1, -1),
                     be0.reshape(1, -1), Wc1, bc1.reshape(1, -1), dinv)

    qs = [_agg_call(h1s[k], src1, dst1, wflat) for k in range(4)]
    out1, ssum1, ssq1 = _asm1_call(qs, h1s, dinv)

    psum, cnt = _pool_call(out1, ssum1, ssq1, g1.reshape(1, -1),
                           be1.reshape(1, -1), batch.reshape(N, 1))
    return _head_call(psum, cnt, Wl2, bl2, Wl1, bl1, Wl0, bl0)
